# baseline (speedup 1.0000x reference)
import functools
import jax
import jax.numpy as jnp
import numpy as np
from jax import lax
from jax.experimental import pallas as pl
from jax.experimental.pallas import tpu as pltpu

N_NODES = 10000
N_EDGES = 160000
NUM_GAUSS = 20
CUTOFF = 10.0

TE = 640
TN = 1000


def _full_spec(shape):
    return pl.BlockSpec(shape, lambda i: tuple(0 for _ in shape))


def _row_spec(block_rows, ncols):
    return pl.BlockSpec((block_rows, ncols), lambda i: (i, 0))


def _ln(x, g, b):
    m = jnp.mean(x, -1, keepdims=True)
    xc = x - m
    v = jnp.mean(xc * xc, -1, keepdims=True)
    return xc * jax.lax.rsqrt(v + 1e-5) * g + b


def _mm(x, w, b=None):
    y = jax.lax.dot_general(x, w, (((1,), (0,)), ((), ())),
                            preferred_element_type=jnp.float32)
    if b is not None:
        y = y + b
    return y


def _node_pre_body(x_ref, wl, bl, w1, b1, w2, b2, wc, bc,
                   h_ref, hn_ref, cent_ref):
    x = x_ref[...]
    h = _mm(x, wl[...], bl[...])
    h_ref[...] = h
    t = jnp.maximum(_mm(h, w1[...], b1[...]), 0.0)
    hn_ref[...] = _mm(t, w2[...], b2[...])
    cent_ref[...] = _mm(h, wc[...], bc[...])


def _edge_nbe_body(pr_ref, pc_ref, ghn_ref, off, we, be, w1, b1, w2, b2,
                   wm, bm, msg_ref):
    vec = pr_ref[...] - pc_ref[...]
    d2 = jnp.sum(vec * vec, -1, keepdims=True) + 1e-8
    d = jnp.sqrt(d2)
    step = CUTOFF / (NUM_GAUSS - 1)
    coeff = -0.5 / step ** 2
    diff = d - off[...]
    smear = jnp.exp(coeff * diff * diff)
    ea = _mm(smear, we[...], be[...])
    t = jnp.maximum(_mm(ea, w1[...], b1[...]), 0.0)
    he = _mm(t, w2[...], b2[...])
    msg_ref[...] = _mm(he * ghn_ref[...], wm[...], bm[...])


def _node_post_body(h_ref, cent_ref, a0_ref, a1_ref, lng, lnb, wo, bo,
                    w1, b1, w2, b2, wc2, bc2,
                    h2_ref, hn2_ref, cent2_ref):
    out = cent_ref[...] + a0_ref[...] + a1_ref[...]
    out = _ln(out, lng[...], lnb[...])
    h2 = h_ref[...] + _mm(jnp.maximum(out, 0.0), wo[...], bo[...])
    h2_ref[...] = h2
    t = jnp.maximum(_mm(h2, w1[...], b1[...]), 0.0)
    hn2_ref[...] = _mm(t, w2[...], b2[...])
    cent2_ref[...] = _mm(h2, wc2[...], bc2[...])


def _edge_bond1_body(hb_ref, gl_ref, gr_ref,
                     wbl, wnl, w1l, b1l, w2l, b2l,
                     wbr, wnr, w1r, b1r, w2r, b2r,
                     wfl, bfl, wfr, bfr, ws, bs,
                     ml_ref, mr_ref, part_ref):
    hb = hb_ref[...]
    gl = gl_ref[...]
    gr = gr_ref[...]
    il = _mm(hb, wbl[...]) * _mm(gl, wnl[...])
    t = jnp.maximum(_mm(il, w1l[...], b1l[...]), 0.0)
    ml_ref[...] = _mm(t, w2l[...], b2l[...])
    ir = _mm(hb, wbr[...]) * _mm(gr, wnr[...])
    t = jnp.maximum(_mm(ir, w1r[...], b1r[...]), 0.0)
    mr_ref[...] = _mm(t, w2r[...], b2r[...])
    part_ref[...] = (_mm(gl, wfl[...], bfl[...]) + _mm(gr, wfr[...], bfr[...])
                     + _mm(hb, ws[...], bs[...]))


def _edge_bond2_body(hb_ref, ga1_ref, ga2_ref, part_ref, ghn2_ref,
                     lng, lnb, wo, bo, w1, b1, w2, b2, wm, bm,
                     hb2_ref, msg2_ref):
    pre = ga1_ref[...] + ga2_ref[...] + part_ref[...]
    pre = _ln(pre, lng[...], lnb[...])
    hb2 = hb_ref[...] + _mm(jnp.maximum(pre, 0.0), wo[...], bo[...])
    hb2_ref[...] = hb2
    t = jnp.maximum(_mm(hb2, w1[...], b1[...]), 0.0)
    he2 = _mm(t, w2[...], b2[...])
    msg2_ref[...] = _mm(he2 * ghn2_ref[...], wm[...], bm[...])


def _node_final_body(h2_ref, cent2_ref, a0_ref, a1_ref, lng, lnb, wo, bo,
                     h3_ref):
    out = cent2_ref[...] + a0_ref[...] + a1_ref[...]
    out = _ln(out, lng[...], lnb[...])
    h3_ref[...] = h2_ref[...] + _mm(jnp.maximum(out, 0.0), wo[...], bo[...])


def _tc_call(body, grid, in_arrs, in_specs, out_shapes, out_specs):
    return pl.pallas_call(
        body,
        grid=(grid,),
        in_specs=in_specs,
        out_specs=out_specs,
        out_shape=out_shapes,
    )(*in_arrs)


def _b2(v):
    return v.reshape(1, -1)


def kernel(h_node, pos_node, h_bond, bond_index, batch, is_mol, is_frag, params):
    P = params
    nbe = P["nbe"][0]
    nbb = P["nbb"][0]
    bb = P["bb"][0]
    row = bond_index[0]
    col = bond_index[1]

    pos_pad = jnp.zeros((N_NODES, 16), jnp.float32).at[:, :3].set(pos_node)
    off = np.zeros((1, 32), np.float32)
    off[0, :NUM_GAUSS] = np.linspace(0.0, CUTOFF, NUM_GAUSS)
    off = jnp.asarray(off)
    we_pad = jnp.zeros((32, 128), jnp.float32).at[:NUM_GAUSS].set(P["edge_emb"]["W"])

    ew = _full_spec

    h, hn_all, cent = _tc_call(
        _node_pre_body, N_NODES // TN,
        [h_node, P["lin_node"]["W"], _b2(P["lin_node"]["b"]),
         nbe["node_net"]["l1"]["W"], _b2(nbe["node_net"]["l1"]["b"]),
         nbe["node_net"]["l2"]["W"], _b2(nbe["node_net"]["l2"]["b"]),
         nbe["centroid"]["W"], _b2(nbe["centroid"]["b"])],
        [_row_spec(TN, 128)] + [ew((128, 128)), ew((1, 128))] * 4,
        [jax.ShapeDtypeStruct((N_NODES, 128), jnp.float32)] * 3,
        [_row_spec(TN, 128)] * 3,
    )

    pr = jnp.take(pos_pad, row, axis=0)
    pc = jnp.take(pos_pad, col, axis=0)
    g_hn = jnp.take(hn_all, col, axis=0)

    (msg,) = _tc_call(
        _edge_nbe_body, N_EDGES // TE,
        [pr, pc, g_hn, off, we_pad, _b2(P["edge_emb"]["b"]),
         nbe["edge_net"]["l1"]["W"], _b2(nbe["edge_net"]["l1"]["b"]),
         nbe["edge_net"]["l2"]["W"], _b2(nbe["edge_net"]["l2"]["b"]),
         nbe["msg_net"]["W"], _b2(nbe["msg_net"]["b"])],
        [_row_spec(TE, 16), _row_spec(TE, 16), _row_spec(TE, 128),
         ew((1, 32)), ew((32, 128)), ew((1, 128)),
         ew((128, 128)), ew((1, 128)), ew((128, 128)), ew((1, 128)),
         ew((128, 128)), ew((1, 128))],
        [jax.ShapeDtypeStruct((N_EDGES, 128), jnp.float32)],
        [_row_spec(TE, 128)],
    )

    aggr = jax.ops.segment_sum(msg, row, num_segments=N_NODES)
    aggr_b = jnp.zeros_like(aggr)

    h2, hn2, cent2 = _tc_call(
        _node_post_body, N_NODES // TN,
        [h, cent, aggr, aggr_b, _b2(nbe["ln_g"]), _b2(nbe["ln_b"]),
         nbe["out"]["W"], _b2(nbe["out"]["b"]),
         nbb["node_net"]["l1"]["W"], _b2(nbb["node_net"]["l1"]["b"]),
         nbb["node_net"]["l2"]["W"], _b2(nbb["node_net"]["l2"]["b"]),
         nbb["centroid"]["W"], _b2(nbb["centroid"]["b"])],
        [_row_spec(TN, 128)] * 4 + [ew((1, 128)), ew((1, 128))]
        + [ew((128, 128)), ew((1, 128))] * 4,
        [jax.ShapeDtypeStruct((N_NODES, 128), jnp.float32)] * 3,
        [_row_spec(TN, 128)] * 3,
    )

    gl = jnp.take(h2, row, axis=0)
    gr = jnp.take(h2, col, axis=0)

    m_l_pre, m_r_pre, part = _tc_call(
        _edge_bond1_body, N_EDGES // TE,
        [h_bond, gl, gr,
         bb["ffn_l"]["bond_lin"]["W"], bb["ffn_l"]["node_lin"]["W"],
         bb["ffn_l"]["inter"]["l1"]["W"], _b2(bb["ffn_l"]["inter"]["l1"]["b"]),
         bb["ffn_l"]["inter"]["l2"]["W"], _b2(bb["ffn_l"]["inter"]["l2"]["b"]),
         bb["ffn_r"]["bond_lin"]["W"], bb["ffn_r"]["node_lin"]["W"],
         bb["ffn_r"]["inter"]["l1"]["W"], _b2(bb["ffn_r"]["inter"]["l1"]["b"]),
         bb["ffn_r"]["inter"]["l2"]["W"], _b2(bb["ffn_r"]["inter"]["l2"]["b"]),
         bb["node_ffn_l"]["W"], _b2(bb["node_ffn_l"]["b"]),
         bb["node_ffn_r"]["W"], _b2(bb["node_ffn_r"]["b"]),
         bb["self_ffn"]["W"], _b2(bb["self_ffn"]["b"])],
        [_row_spec(TE, 128)] * 3
        + [ew((128, 256)), ew((128, 256)), ew((256, 256)), ew((1, 256)),
           ew((256, 128)), ew((1, 128))] * 2
        + [ew((128, 128)), ew((1, 128))] * 3,
        [jax.ShapeDtypeStruct((N_EDGES, 128), jnp.float32)] * 3,
        [_row_spec(TE, 128)] * 3,
    )

    A1 = jax.ops.segment_sum(m_l_pre, col, num_segments=N_NODES)
    A2 = jax.ops.segment_sum(m_r_pre, row, num_segments=N_NODES)

    gA1 = jnp.take(A1, row, axis=0)
    gA2 = jnp.take(A2, col, axis=0)
    g_hn2 = jnp.take(hn2, col, axis=0)

    hb2, msg2 = _tc_call(
        _edge_bond2_body, N_EDGES // TE,
        [h_bond, gA1, gA2, part, g_hn2,
         _b2(bb["ln_g"]), _b2(bb["ln_b"]),
         bb["out"]["W"], _b2(bb["out"]["b"]),
         nbb["edge_net"]["l1"]["W"], _b2(nbb["edge_net"]["l1"]["b"]),
         nbb["edge_net"]["l2"]["W"], _b2(nbb["edge_net"]["l2"]["b"]),
         nbb["msg_net"]["W"], _b2(nbb["msg_net"]["b"])],
        [_row_spec(TE, 128)] * 5 + [ew((1, 128)), ew((1, 128))]
        + [ew((128, 128)), ew((1, 128))] * 4,
        [jax.ShapeDtypeStruct((N_EDGES, 128), jnp.float32)] * 2,
        [_row_spec(TE, 128)] * 2,
    )

    aggr2 = jax.ops.segment_sum(msg2, row, num_segments=N_NODES)
    aggr2_b = jnp.zeros_like(aggr2)

    (h3,) = _tc_call(
        _node_final_body, N_NODES // TN,
        [h2, cent2, aggr2, aggr2_b, _b2(nbb["ln_g"]), _b2(nbb["ln_b"]),
         nbb["out"]["W"], _b2(nbb["out"]["b"])],
        [_row_spec(TN, 128)] * 4 + [ew((1, 128)), ew((1, 128)),
                                    ew((128, 128)), ew((1, 128))],
        [jax.ShapeDtypeStruct((N_NODES, 128), jnp.float32)],
        [_row_spec(TN, 128)],
    )

    return h3, hb2


# speedup vs baseline: 2.5223x; 2.5223x over previous
import functools
import jax
import jax.numpy as jnp
import numpy as np
from jax import lax
from jax.experimental import pallas as pl
from jax.experimental.pallas import tpu as pltpu
from jax.experimental.pallas import tpu_sc as plsc

N_NODES = 10000
N_EDGES = 160000
NUM_GAUSS = 20
CUTOFF = 10.0

TE = 640
TN = 1000

NC, NS = 2, 16
NW = NC * NS
CH = 128
NCHUNK = N_EDGES // CH
TRIPS32 = -(-NCHUNK // NW)
TRIPS16 = -(-NCHUNK // NS)
N_PAD = 10240
NODE_SLICE = N_PAD // NS

_SC_MESH = plsc.VectorSubcoreMesh(core_axis_name="c", subcore_axis_name="s")


def _make_geom_gather():
    scratch = [pltpu.VMEM((N_NODES * 4,), jnp.float32),
               pltpu.VMEM((CH,), jnp.int32),
               pltpu.VMEM((CH,), jnp.int32),
               pltpu.VMEM((CH, 128), jnp.float32),
               pltpu.VMEM((CH,), jnp.float32),
               pltpu.SemaphoreType.DMA]
    out_type = [jax.ShapeDtypeStruct((N_EDGES, 128), jnp.float32),
                jax.ShapeDtypeStruct((N_EDGES,), jnp.float32)]

    @functools.partial(pl.kernel, out_type=out_type, mesh=_SC_MESH,
                       scratch_types=scratch,
                       compiler_params=pltpu.CompilerParams(
                           needs_layout_passes=False))
    def gk(hn_tab, pos4, row, col, ghn_out, d2_out,
           pos_v, ridx_v, cidx_v, rows_v, d2_v, sem):
        wid = lax.axis_index("s") * NC + lax.axis_index("c")
        pltpu.sync_copy(pos4, pos_v)

        def body(i, carry):
            chunk = wid + i * NW

            @pl.when(chunk < NCHUNK)
            def _():
                base = chunk * CH
                pltpu.sync_copy(row.at[pl.ds(base, CH)], ridx_v)
                pltpu.sync_copy(col.at[pl.ds(base, CH)], cidx_v)
                cp = pltpu.async_copy(hn_tab.at[cidx_v], rows_v, sem)
                for t in range(CH // 16):
                    ri = ridx_v[pl.ds(t * 16, 16)] * 4
                    ci = cidx_v[pl.ds(t * 16, 16)] * 4
                    dx = (plsc.load_gather(pos_v, [ri])
                          - plsc.load_gather(pos_v, [ci]))
                    dy = (plsc.load_gather(pos_v, [ri + 1])
                          - plsc.load_gather(pos_v, [ci + 1]))
                    dz = (plsc.load_gather(pos_v, [ri + 2])
                          - plsc.load_gather(pos_v, [ci + 2]))
                    d2_v[pl.ds(t * 16, 16)] = (dx * dx + dy * dy + dz * dz
                                               + 1e-8)
                pltpu.sync_copy(d2_v, d2_out.at[pl.ds(base, CH)])
                cp.wait()
                pltpu.sync_copy(rows_v, ghn_out.at[pl.ds(base, CH)])
            return carry

        lax.fori_loop(0, TRIPS32, body, 0)

    return gk


def _make_gather(dims):
    k = len(dims)
    scratch = []
    for d in dims:
        scratch.append(pltpu.VMEM((CH,), jnp.int32))
        scratch.append(pltpu.VMEM((CH, d), jnp.float32))
        scratch.append(pltpu.SemaphoreType.DMA)
    out_type = [jax.ShapeDtypeStruct((N_EDGES, d), jnp.float32) for d in dims]

    @functools.partial(pl.kernel, out_type=out_type, mesh=_SC_MESH,
                       scratch_types=scratch)
    def gk(*refs):
        tabs = refs[0:2 * k:2]
        idxs = refs[1:2 * k:2]
        outs = refs[2 * k:3 * k]
        scr = refs[3 * k:]
        wid = lax.axis_index("s") * NC + lax.axis_index("c")

        def body(i, carry):
            chunk = wid + i * NW

            @pl.when(chunk < NCHUNK)
            def _():
                base = chunk * CH
                for j in range(k):
                    idx_v, rows_v, sem = scr[3 * j], scr[3 * j + 1], scr[3 * j + 2]
                    pltpu.sync_copy(idxs[j].at[pl.ds(base, CH)], idx_v)
                    pltpu.async_copy(tabs[j].at[idx_v], rows_v, sem).wait()
                    pltpu.sync_copy(rows_v, outs[j].at[pl.ds(base, CH)])
            return carry

        lax.fori_loop(0, TRIPS32, body, 0)

    return gk


def _make_scatter1():
    scratch = [pltpu.VMEM((TRIPS32, CH), jnp.int32),
               pltpu.VMEM((CH, 128), jnp.float32),
               pltpu.VMEM_SHARED((N_PAD, 128), jnp.float32)]
    out_type = jax.ShapeDtypeStruct((NC, N_PAD, 128), jnp.float32)

    @functools.partial(pl.kernel, out_type=out_type, mesh=_SC_MESH,
                       scratch_types=scratch)
    def sk(vals, idx, zeros, out, idx_v, rows_v, accum):
        c = lax.axis_index("c")
        s = lax.axis_index("s")
        wid = s * NC + c
        base_n = s * NODE_SLICE
        pltpu.sync_copy(zeros.at[pl.ds(base_n, NODE_SLICE)],
                        accum.at[pl.ds(base_n, NODE_SLICE)])

        def stage(i, carry):
            chunk = wid + i * NW

            @pl.when(chunk < NCHUNK)
            def _():
                pltpu.sync_copy(idx.at[pl.ds(chunk * CH, CH)], idx_v.at[i])
            return carry

        lax.fori_loop(0, TRIPS32, stage, 0)
        plsc.subcore_barrier()

        def body(i, carry):
            chunk = wid + i * NW

            @pl.when(chunk < NCHUNK)
            def _():
                pltpu.sync_copy(vals.at[pl.ds(chunk * CH, CH)], rows_v)
                pltpu.sync_copy(rows_v, accum.at[idx_v.at[i]], add=True)
            return carry

        lax.fori_loop(0, TRIPS32, body, 0)
        plsc.subcore_barrier()
        pltpu.sync_copy(accum.at[pl.ds(base_n, NODE_SLICE)],
                        out.at[c, pl.ds(base_n, NODE_SLICE)])

    return sk


def _make_scatter2():
    scratch = [pltpu.VMEM((TRIPS16, CH), jnp.int32),
               pltpu.VMEM((CH, 128), jnp.float32),
               pltpu.VMEM_SHARED((N_PAD, 128), jnp.float32)]
    out_type = jax.ShapeDtypeStruct((NC, N_PAD, 128), jnp.float32)

    @functools.partial(pl.kernel, out_type=out_type, mesh=_SC_MESH,
                       scratch_types=scratch)
    def sk(vl, il, vr, ir, zeros, out, idx_v, rows_v, accum):
        c = lax.axis_index("c")
        s = lax.axis_index("s")
        base_n = s * NODE_SLICE
        pltpu.sync_copy(zeros.at[pl.ds(base_n, NODE_SLICE)],
                        accum.at[pl.ds(base_n, NODE_SLICE)])

        def run(vals, idx):
            def stage(i, carry):
                chunk = s + i * NS

                @pl.when(chunk < NCHUNK)
                def _():
                    pltpu.sync_copy(idx.at[pl.ds(chunk * CH, CH)], idx_v.at[i])
                return carry

            lax.fori_loop(0, TRIPS16, stage, 0)
            plsc.subcore_barrier()

            def body(i, carry):
                chunk = s + i * NS

                @pl.when(chunk < NCHUNK)
                def _():
                    pltpu.sync_copy(vals.at[pl.ds(chunk * CH, CH)], rows_v)
                    pltpu.sync_copy(rows_v, accum.at[idx_v.at[i]], add=True)
                return carry

            lax.fori_loop(0, TRIPS16, body, 0)
            plsc.subcore_barrier()
            pltpu.sync_copy(accum.at[pl.ds(base_n, NODE_SLICE)],
                            out.at[c, pl.ds(base_n, NODE_SLICE)])

        @pl.when(c == 0)
        def _():
            run(vl, il)

        @pl.when(c == 1)
        def _():
            run(vr, ir)

    return sk


def _full_spec(shape):
    return pl.BlockSpec(shape, lambda i: tuple(0 for _ in shape))


def _row_spec(block_rows, ncols):
    return pl.BlockSpec((block_rows, ncols), lambda i: (i, 0))


def _ln(x, g, b):
    m = jnp.mean(x, -1, keepdims=True)
    xc = x - m
    v = jnp.mean(xc * xc, -1, keepdims=True)
    return xc * jax.lax.rsqrt(v + 1e-5) * g + b


def _mm(x, w, b=None):
    y = jax.lax.dot_general(x, w, (((1,), (0,)), ((), ())),
                            preferred_element_type=jnp.float32)
    if b is not None:
        y = y + b
    return y


def _node_pre_body(x_ref, wl, bl, w1, b1, w2, b2, wc, bc,
                   h_ref, hn_ref, cent_ref):
    x = x_ref[...]
    h = _mm(x, wl[...], bl[...])
    h_ref[...] = h
    t = jnp.maximum(_mm(h, w1[...], b1[...]), 0.0)
    hn_ref[...] = _mm(t, w2[...], b2[...])
    cent_ref[...] = _mm(h, wc[...], bc[...])


def _edge_nbe_body(d2_ref, ghn_ref, off, we, be, w1, b1, w2, b2,
                   wm, bm, msg_ref):
    ghn = ghn_ref[...]
    d = jnp.sqrt(d2_ref[...])
    step = CUTOFF / (NUM_GAUSS - 1)
    coeff = -0.5 / step ** 2
    diff = d - off[...]
    smear = jnp.exp(coeff * diff * diff)
    ea = _mm(smear, we[...], be[...])
    t = jnp.maximum(_mm(ea, w1[...], b1[...]), 0.0)
    he = _mm(t, w2[...], b2[...])
    msg_ref[...] = _mm(he * ghn, wm[...], bm[...])


def _node_post_body(h_ref, cent_ref, agg_ref, lng, lnb, wo, bo,
                    w1, b1, w2, b2, wc2, bc2,
                    h2_ref, hn2_ref, cent2_ref):
    out = cent_ref[...] + agg_ref[0] + agg_ref[1]
    out = _ln(out, lng[...], lnb[...])
    h2 = h_ref[...] + _mm(jnp.maximum(out, 0.0), wo[...], bo[...])
    h2_ref[...] = h2
    t = jnp.maximum(_mm(h2, w1[...], b1[...]), 0.0)
    hn2_ref[...] = _mm(t, w2[...], b2[...])
    cent2_ref[...] = _mm(h2, wc2[...], bc2[...])


def _edge_bond1_body(hb_ref, gl_ref, gr_ref,
                     wbl, wnl, w1l, b1l, w2l, b2l,
                     wbr, wnr, w1r, b1r, w2r, b2r,
                     wfl, bfl, wfr, bfr, ws, bs,
                     ml_ref, mr_ref, part_ref):
    hb = hb_ref[...]
    gl = gl_ref[...]
    gr = gr_ref[...]
    il = _mm(hb, wbl[...]) * _mm(gl, wnl[...])
    t = jnp.maximum(_mm(il, w1l[...], b1l[...]), 0.0)
    ml_ref[...] = _mm(t, w2l[...], b2l[...])
    ir = _mm(hb, wbr[...]) * _mm(gr, wnr[...])
    t = jnp.maximum(_mm(ir, w1r[...], b1r[...]), 0.0)
    mr_ref[...] = _mm(t, w2r[...], b2r[...])
    part_ref[...] = (_mm(gl, wfl[...], bfl[...]) + _mm(gr, wfr[...], bfr[...])
                     + _mm(hb, ws[...], bs[...]))


def _edge_bond2_body(hb_ref, ga1_ref, gac_ref, part_ref,
                     lng, lnb, wo, bo, w1, b1, w2, b2, wm, bm,
                     hb2_ref, msg2_ref):
    gac = gac_ref[...]
    ga2 = gac[:, :128]
    ghn2 = gac[:, 128:]
    pre = ga1_ref[...] + ga2 + part_ref[...]
    pre = _ln(pre, lng[...], lnb[...])
    hb2 = hb_ref[...] + _mm(jnp.maximum(pre, 0.0), wo[...], bo[...])
    hb2_ref[...] = hb2
    t = jnp.maximum(_mm(hb2, w1[...], b1[...]), 0.0)
    he2 = _mm(t, w2[...], b2[...])
    msg2_ref[...] = _mm(he2 * ghn2, wm[...], bm[...])


def _node_final_body(h2_ref, cent2_ref, agg_ref, lng, lnb, wo, bo,
                     h3_ref):
    out = cent2_ref[...] + agg_ref[0] + agg_ref[1]
    out = _ln(out, lng[...], lnb[...])
    h3_ref[...] = h2_ref[...] + _mm(jnp.maximum(out, 0.0), wo[...], bo[...])


def _tc_call(body, grid, in_arrs, in_specs, out_shapes, out_specs):
    return pl.pallas_call(
        body,
        grid=(grid,),
        in_specs=in_specs,
        out_specs=out_specs,
        out_shape=out_shapes,
    )(*in_arrs)


def _agg_spec():
    return pl.BlockSpec((2, TN, 128), lambda i: (0, i, 0))


_gather_g12 = _make_geom_gather()
_gather_g3 = _make_gather((128, 128))
_gather_g4 = _make_gather((128, 256))
_scatter_s1 = _make_scatter1()
_scatter_s2 = _make_scatter2()


def _b2(v):
    return v.reshape(1, -1)


def kernel(h_node, pos_node, h_bond, bond_index, batch, is_mol, is_frag, params):
    P = params
    nbe = P["nbe"][0]
    nbb = P["nbb"][0]
    bb = P["bb"][0]
    row = bond_index[0]
    col = bond_index[1]
    zeros_n = jnp.zeros((N_PAD, 128), jnp.float32)

    pos4 = jnp.zeros((N_NODES, 4), jnp.float32).at[:, :3].set(pos_node)
    pos4 = pos4.reshape(N_NODES * 4)
    off = np.zeros((1, 32), np.float32)
    off[0, :NUM_GAUSS] = np.linspace(0.0, CUTOFF, NUM_GAUSS)
    off = jnp.asarray(off)
    we_pad = jnp.zeros((32, 128), jnp.float32).at[:NUM_GAUSS].set(P["edge_emb"]["W"])

    ew = _full_spec

    h, hn_all, cent = _tc_call(
        _node_pre_body, N_NODES // TN,
        [h_node, P["lin_node"]["W"], _b2(P["lin_node"]["b"]),
         nbe["node_net"]["l1"]["W"], _b2(nbe["node_net"]["l1"]["b"]),
         nbe["node_net"]["l2"]["W"], _b2(nbe["node_net"]["l2"]["b"]),
         nbe["centroid"]["W"], _b2(nbe["centroid"]["b"])],
        [_row_spec(TN, 128)] + [ew((128, 128)), ew((1, 128))] * 4,
        [jax.ShapeDtypeStruct((N_NODES, 128), jnp.float32)] * 3,
        [_row_spec(TN, 128)] * 3,
    )

    ghn, d2 = _gather_g12(hn_all, pos4, row, col)
    d2 = d2.reshape(N_EDGES, 1)

    (msg,) = _tc_call(
        _edge_nbe_body, N_EDGES // TE,
        [d2, ghn, off, we_pad, _b2(P["edge_emb"]["b"]),
         nbe["edge_net"]["l1"]["W"], _b2(nbe["edge_net"]["l1"]["b"]),
         nbe["edge_net"]["l2"]["W"], _b2(nbe["edge_net"]["l2"]["b"]),
         nbe["msg_net"]["W"], _b2(nbe["msg_net"]["b"])],
        [_row_spec(TE, 1), _row_spec(TE, 128),
         ew((1, 32)), ew((32, 128)), ew((1, 128)),
         ew((128, 128)), ew((1, 128)), ew((128, 128)), ew((1, 128)),
         ew((128, 128)), ew((1, 128))],
        [jax.ShapeDtypeStruct((N_EDGES, 128), jnp.float32)],
        [_row_spec(TE, 128)],
    )

    aggr = _scatter_s1(msg, row, zeros_n)

    h2, hn2, cent2 = _tc_call(
        _node_post_body, N_NODES // TN,
        [h, cent, aggr, _b2(nbe["ln_g"]), _b2(nbe["ln_b"]),
         nbe["out"]["W"], _b2(nbe["out"]["b"]),
         nbb["node_net"]["l1"]["W"], _b2(nbb["node_net"]["l1"]["b"]),
         nbb["node_net"]["l2"]["W"], _b2(nbb["node_net"]["l2"]["b"]),
         nbb["centroid"]["W"], _b2(nbb["centroid"]["b"])],
        [_row_spec(TN, 128)] * 2 + [_agg_spec()]
        + [ew((1, 128)), ew((1, 128))]
        + [ew((128, 128)), ew((1, 128))] * 4,
        [jax.ShapeDtypeStruct((N_NODES, 128), jnp.float32)] * 3,
        [_row_spec(TN, 128)] * 3,
    )

    gl, gr = _gather_g3(h2, row, h2, col)

    m_l_pre, m_r_pre, part = _tc_call(
        _edge_bond1_body, N_EDGES // TE,
        [h_bond, gl, gr,
         bb["ffn_l"]["bond_lin"]["W"], bb["ffn_l"]["node_lin"]["W"],
         bb["ffn_l"]["inter"]["l1"]["W"], _b2(bb["ffn_l"]["inter"]["l1"]["b"]),
         bb["ffn_l"]["inter"]["l2"]["W"], _b2(bb["ffn_l"]["inter"]["l2"]["b"]),
         bb["ffn_r"]["bond_lin"]["W"], bb["ffn_r"]["node_lin"]["W"],
         bb["ffn_r"]["inter"]["l1"]["W"], _b2(bb["ffn_r"]["inter"]["l1"]["b"]),
         bb["ffn_r"]["inter"]["l2"]["W"], _b2(bb["ffn_r"]["inter"]["l2"]["b"]),
         bb["node_ffn_l"]["W"], _b2(bb["node_ffn_l"]["b"]),
         bb["node_ffn_r"]["W"], _b2(bb["node_ffn_r"]["b"]),
         bb["self_ffn"]["W"], _b2(bb["self_ffn"]["b"])],
        [_row_spec(TE, 128)] * 3
        + [ew((128, 256)), ew((128, 256)), ew((256, 256)), ew((1, 256)),
           ew((256, 128)), ew((1, 128))] * 2
        + [ew((128, 128)), ew((1, 128))] * 3,
        [jax.ShapeDtypeStruct((N_EDGES, 128), jnp.float32)] * 3,
        [_row_spec(TE, 128)] * 3,
    )

    A12 = _scatter_s2(m_l_pre, col, m_r_pre, row, zeros_n)
    A1 = A12[0]
    tac = jnp.concatenate([A12[1][:N_NODES], hn2], axis=1)

    gA1, gac = _gather_g4(A1, row, tac, col)

    hb2, msg2 = _tc_call(
        _edge_bond2_body, N_EDGES // TE,
        [h_bond, gA1, gac, part,
         _b2(bb["ln_g"]), _b2(bb["ln_b"]),
         bb["out"]["W"], _b2(bb["out"]["b"]),
         nbb["edge_net"]["l1"]["W"], _b2(nbb["edge_net"]["l1"]["b"]),
         nbb["edge_net"]["l2"]["W"], _b2(nbb["edge_net"]["l2"]["b"]),
         nbb["msg_net"]["W"], _b2(nbb["msg_net"]["b"])],
        [_row_spec(TE, 128), _row_spec(TE, 128), _row_spec(TE, 256),
         _row_spec(TE, 128)]
        + [ew((1, 128)), ew((1, 128))]
        + [ew((128, 128)), ew((1, 128))] * 4,
        [jax.ShapeDtypeStruct((N_EDGES, 128), jnp.float32)] * 2,
        [_row_spec(TE, 128)] * 2,
    )

    aggr2 = _scatter_s1(msg2, row, zeros_n)

    (h3,) = _tc_call(
        _node_final_body, N_NODES // TN,
        [h2, cent2, aggr2, _b2(nbb["ln_g"]), _b2(nbb["ln_b"]),
         nbb["out"]["W"], _b2(nbb["out"]["b"])],
        [_row_spec(TN, 128)] * 2 + [_agg_spec()]
        + [ew((1, 128)), ew((1, 128)), ew((128, 128)), ew((1, 128))],
        [jax.ShapeDtypeStruct((N_NODES, 128), jnp.float32)],
        [_row_spec(TN, 128)],
    )

    return h3, hb2


# speedup vs baseline: 3.2116x; 1.2733x over previous
import functools
import jax
import jax.numpy as jnp
import numpy as np
from jax import lax
from jax.experimental import pallas as pl
from jax.experimental.pallas import tpu as pltpu
from jax.experimental.pallas import tpu_sc as plsc

N_NODES = 10000
N_EDGES = 160000
NUM_GAUSS = 20
CUTOFF = 10.0

TE = 640
TN = 1000

NC, NS = 2, 16
NW = NC * NS
CH = 128
NCHUNK = N_EDGES // CH
TRIPS32 = -(-NCHUNK // NW)
TRIPS16 = -(-NCHUNK // NS)
N_PAD = 10240
NODE_SLICE = N_PAD // NS

_SC_MESH = plsc.VectorSubcoreMesh(core_axis_name="c", subcore_axis_name="s")


def _valid(wid, t, ways):
    return jnp.logical_and(t >= 0, wid + t * ways < NCHUNK)


def _wait(src, dst, sem):
    pltpu.make_async_copy(src, dst, sem).wait()


def _make_gather(dims):
    k = len(dims)
    scratch = []
    for d in dims:
        scratch.append(pltpu.VMEM((TRIPS32, CH), jnp.int32))
        for b in range(2):
            scratch.append(pltpu.VMEM((CH, d), jnp.float32))
            scratch.append(pltpu.SemaphoreType.DMA)
            scratch.append(pltpu.SemaphoreType.DMA)
    out_type = [jax.ShapeDtypeStruct((N_EDGES, d), jnp.float32) for d in dims]

    @functools.partial(pl.kernel, out_type=out_type, mesh=_SC_MESH,
                       scratch_types=scratch)
    def gk(*refs):
        tabs = refs[0:2 * k:2]
        idx3 = refs[1:2 * k:2]
        outs = refs[2 * k:3 * k]
        scr = refs[3 * k:]
        idx_all = [scr[7 * j] for j in range(k)]
        rows = [[scr[7 * j + 1 + 3 * b] for b in range(2)] for j in range(k)]
        semg = [[scr[7 * j + 2 + 3 * b] for b in range(2)] for j in range(k)]
        semw = [[scr[7 * j + 3 + 3 * b] for b in range(2)] for j in range(k)]
        wid = lax.axis_index("s") * NC + lax.axis_index("c")
        for j in range(k):
            pltpu.sync_copy(idx3[j].at[:, wid], idx_all[j])

        def trip(t, b):
            tm1, tm2 = t - 1, t - 2
            for j in range(k):
                @pl.when(_valid(wid, tm2, NW))
                def _(j=j):
                    _wait(rows[j][b], outs[j].at[pl.ds(0, CH)], semw[j][b])
            for j in range(k):
                @pl.when(_valid(wid, t, NW))
                def _(j=j):
                    pltpu.async_copy(tabs[j].at[idx_all[j].at[t]],
                                     rows[j][b], semg[j][b])
            for j in range(k):
                @pl.when(_valid(wid, tm1, NW))
                def _(j=j):
                    bp = 1 - b
                    _wait(tabs[j].at[pl.ds(0, CH)], rows[j][bp], semg[j][bp])
                    base = (wid + tm1 * NW) * CH
                    pltpu.async_copy(rows[j][bp], outs[j].at[pl.ds(base, CH)],
                                     semw[j][bp])

        def pair(i, carry):
            trip(2 * i, 0)
            trip(2 * i + 1, 1)
            return carry

        lax.fori_loop(0, (TRIPS32 + 3) // 2, pair, 0)

    return gk


def _make_geom_gather():
    scratch = [pltpu.VMEM((N_NODES * 4,), jnp.float32),
               pltpu.VMEM((TRIPS32, CH), jnp.int32),
               pltpu.VMEM((TRIPS32, CH), jnp.int32),
               pltpu.VMEM((CH, 128), jnp.float32),
               pltpu.VMEM((CH, 128), jnp.float32),
               pltpu.VMEM((CH,), jnp.float32),
               pltpu.SemaphoreType.DMA,
               pltpu.SemaphoreType.DMA,
               pltpu.SemaphoreType.DMA,
               pltpu.SemaphoreType.DMA]
    out_type = [jax.ShapeDtypeStruct((N_EDGES, 128), jnp.float32),
                jax.ShapeDtypeStruct((N_EDGES,), jnp.float32)]

    @functools.partial(pl.kernel, out_type=out_type, mesh=_SC_MESH,
                       scratch_types=scratch,
                       compiler_params=pltpu.CompilerParams(
                           needs_layout_passes=False))
    def gk(hn_tab, pos4, row3, col3, ghn_out, d2_out,
           pos_v, ridx_all, cidx_all, rows0, rows1, d2_v,
           semg0, semg1, semw0, semw1):
        rows = [rows0, rows1]
        semg = [semg0, semg1]
        semw = [semw0, semw1]
        wid = lax.axis_index("s") * NC + lax.axis_index("c")
        pltpu.sync_copy(pos4, pos_v)
        pltpu.sync_copy(row3.at[:, wid], ridx_all)
        pltpu.sync_copy(col3.at[:, wid], cidx_all)

        def trip(t, b):
            tm1, tm2 = t - 1, t - 2

            @pl.when(_valid(wid, tm2, NW))
            def _():
                _wait(rows[b], ghn_out.at[pl.ds(0, CH)], semw[b])

            @pl.when(_valid(wid, t, NW))
            def _():
                pltpu.async_copy(hn_tab.at[cidx_all.at[t]], rows[b], semg[b])
                for l in range(CH // 16):
                    ri = ridx_all[t, pl.ds(l * 16, 16)] * 4
                    ci = cidx_all[t, pl.ds(l * 16, 16)] * 4
                    dx = (plsc.load_gather(pos_v, [ri])
                          - plsc.load_gather(pos_v, [ci]))
                    dy = (plsc.load_gather(pos_v, [ri + 1])
                          - plsc.load_gather(pos_v, [ci + 1]))
                    dz = (plsc.load_gather(pos_v, [ri + 2])
                          - plsc.load_gather(pos_v, [ci + 2]))
                    d2_v[pl.ds(l * 16, 16)] = (dx * dx + dy * dy + dz * dz
                                               + 1e-8)
                pltpu.sync_copy(d2_v,
                                d2_out.at[pl.ds((wid + t * NW) * CH, CH)])

            @pl.when(_valid(wid, tm1, NW))
            def _():
                bp = 1 - b
                _wait(hn_tab.at[pl.ds(0, CH)], rows[bp], semg[bp])
                base = (wid + tm1 * NW) * CH
                pltpu.async_copy(rows[bp], ghn_out.at[pl.ds(base, CH)],
                                 semw[bp])

        def pair(i, carry):
            trip(2 * i, 0)
            trip(2 * i + 1, 1)
            return carry

        lax.fori_loop(0, (TRIPS32 + 3) // 2, pair, 0)

    return gk


def _scatter_loop(vals, idx_all, accum, rows, semv, wid, ways, trips):
    def fire(t, b):
        @pl.when(_valid(wid, t, ways))
        def _():
            base = (wid + t * ways) * CH
            pltpu.async_copy(vals.at[pl.ds(base, CH)], rows[b], semv[b])

    fire(0, 0)
    fire(1, 1)

    def trip(t, b):
        @pl.when(_valid(wid, t, ways))
        def _():
            _wait(vals.at[pl.ds(0, CH)], rows[b], semv[b])
            pltpu.sync_copy(rows[b], accum.at[idx_all.at[t]], add=True)
        fire(t + 2, b)

    def pair(i, carry):
        trip(2 * i, 0)
        trip(2 * i + 1, 1)
        return carry

    lax.fori_loop(0, (trips + 1) // 2, pair, 0)


def _make_scatter1():
    scratch = [pltpu.VMEM((TRIPS32, CH), jnp.int32),
               pltpu.VMEM((CH, 128), jnp.float32),
               pltpu.VMEM((CH, 128), jnp.float32),
               pltpu.SemaphoreType.DMA,
               pltpu.SemaphoreType.DMA,
               pltpu.VMEM_SHARED((N_PAD, 128), jnp.float32)]
    out_type = jax.ShapeDtypeStruct((NC, N_PAD, 128), jnp.float32)

    @functools.partial(pl.kernel, out_type=out_type, mesh=_SC_MESH,
                       scratch_types=scratch)
    def sk(vals, idx3, zeros, out, idx_all, rows0, rows1, semv0, semv1, accum):
        c = lax.axis_index("c")
        s = lax.axis_index("s")
        wid = s * NC + c
        base_n = s * NODE_SLICE
        pltpu.sync_copy(zeros.at[pl.ds(base_n, NODE_SLICE)],
                        accum.at[pl.ds(base_n, NODE_SLICE)])
        pltpu.sync_copy(idx3.at[:, wid], idx_all)
        plsc.subcore_barrier()
        _scatter_loop(vals, idx_all, accum, [rows0, rows1],
                      [semv0, semv1], wid, NW, TRIPS32)
        plsc.subcore_barrier()
        pltpu.sync_copy(accum.at[pl.ds(base_n, NODE_SLICE)],
                        out.at[c, pl.ds(base_n, NODE_SLICE)])

    return sk


def _make_scatter2():
    scratch = [pltpu.VMEM((TRIPS16, CH), jnp.int32),
               pltpu.VMEM((CH, 128), jnp.float32),
               pltpu.VMEM((CH, 128), jnp.float32),
               pltpu.SemaphoreType.DMA,
               pltpu.SemaphoreType.DMA,
               pltpu.VMEM_SHARED((N_PAD, 128), jnp.float32)]
    out_type = jax.ShapeDtypeStruct((NC, N_PAD, 128), jnp.float32)

    @functools.partial(pl.kernel, out_type=out_type, mesh=_SC_MESH,
                       scratch_types=scratch)
    def sk(vl, il3, vr, ir3, zeros, out, idx_all, rows0, rows1,
           semv0, semv1, accum):
        c = lax.axis_index("c")
        s = lax.axis_index("s")
        base_n = s * NODE_SLICE
        pltpu.sync_copy(zeros.at[pl.ds(base_n, NODE_SLICE)],
                        accum.at[pl.ds(base_n, NODE_SLICE)])

        def run(vals, idx3):
            pltpu.sync_copy(idx3.at[:, s], idx_all)
            plsc.subcore_barrier()
            _scatter_loop(vals, idx_all, accum, [rows0, rows1],
                          [semv0, semv1], s, NS, TRIPS16)
            plsc.subcore_barrier()
            pltpu.sync_copy(accum.at[pl.ds(base_n, NODE_SLICE)],
                            out.at[c, pl.ds(base_n, NODE_SLICE)])

        @pl.when(c == 0)
        def _():
            run(vl, il3)

        @pl.when(c == 1)
        def _():
            run(vr, ir3)

    return sk


def _full_spec(shape):
    return pl.BlockSpec(shape, lambda i: tuple(0 for _ in shape))


def _row_spec(block_rows, ncols):
    return pl.BlockSpec((block_rows, ncols), lambda i: (i, 0))


def _ln(x, g, b):
    m = jnp.mean(x, -1, keepdims=True)
    xc = x - m
    v = jnp.mean(xc * xc, -1, keepdims=True)
    return xc * jax.lax.rsqrt(v + 1e-5) * g + b


def _mm(x, w, b=None):
    y = jax.lax.dot_general(x, w, (((1,), (0,)), ((), ())),
                            preferred_element_type=jnp.float32)
    if b is not None:
        y = y + b
    return y


def _node_pre_body(x_ref, wl, bl, w1, b1, w2, b2, wc, bc,
                   h_ref, hn_ref, cent_ref):
    x = x_ref[...]
    h = _mm(x, wl[...], bl[...])
    h_ref[...] = h
    t = jnp.maximum(_mm(h, w1[...], b1[...]), 0.0)
    hn_ref[...] = _mm(t, w2[...], b2[...])
    cent_ref[...] = _mm(h, wc[...], bc[...])


def _edge_nbe_body(d2_ref, ghn_ref, off, we, be, w1, b1, w2, b2,
                   wm, bm, msg_ref):
    ghn = ghn_ref[...]
    d = jnp.sqrt(d2_ref[...])
    step = CUTOFF / (NUM_GAUSS - 1)
    coeff = -0.5 / step ** 2
    diff = d - off[...]
    smear = jnp.exp(coeff * diff * diff)
    ea = _mm(smear, we[...], be[...])
    t = jnp.maximum(_mm(ea, w1[...], b1[...]), 0.0)
    he = _mm(t, w2[...], b2[...])
    msg_ref[...] = _mm(he * ghn, wm[...], bm[...])


def _node_post_body(h_ref, cent_ref, agg_ref, lng, lnb, wo, bo,
                    w1, b1, w2, b2, wc2, bc2,
                    h2_ref, hn2_ref, cent2_ref):
    out = cent_ref[...] + agg_ref[0] + agg_ref[1]
    out = _ln(out, lng[...], lnb[...])
    h2 = h_ref[...] + _mm(jnp.maximum(out, 0.0), wo[...], bo[...])
    h2_ref[...] = h2
    t = jnp.maximum(_mm(h2, w1[...], b1[...]), 0.0)
    hn2_ref[...] = _mm(t, w2[...], b2[...])
    cent2_ref[...] = _mm(h2, wc2[...], bc2[...])


def _edge_bond1_body(hb_ref, gl_ref, gr_ref,
                     wbl, wnl, w1l, b1l, w2l, b2l,
                     wbr, wnr, w1r, b1r, w2r, b2r,
                     wfl, bfl, wfr, bfr, ws, bs,
                     ml_ref, mr_ref, part_ref):
    hb = hb_ref[...]
    gl = gl_ref[...]
    gr = gr_ref[...]
    il = _mm(hb, wbl[...]) * _mm(gl, wnl[...])
    t = jnp.maximum(_mm(il, w1l[...], b1l[...]), 0.0)
    ml_ref[...] = _mm(t, w2l[...], b2l[...])
    ir = _mm(hb, wbr[...]) * _mm(gr, wnr[...])
    t = jnp.maximum(_mm(ir, w1r[...], b1r[...]), 0.0)
    mr_ref[...] = _mm(t, w2r[...], b2r[...])
    part_ref[...] = (_mm(gl, wfl[...], bfl[...]) + _mm(gr, wfr[...], bfr[...])
                     + _mm(hb, ws[...], bs[...]))


def _edge_bond2_body(hb_ref, ga1_ref, gac_ref, part_ref,
                     lng, lnb, wo, bo, w1, b1, w2, b2, wm, bm,
                     hb2_ref, msg2_ref):
    gac = gac_ref[...]
    ga2 = gac[:, :128]
    ghn2 = gac[:, 128:]
    pre = ga1_ref[...] + ga2 + part_ref[...]
    pre = _ln(pre, lng[...], lnb[...])
    hb2 = hb_ref[...] + _mm(jnp.maximum(pre, 0.0), wo[...], bo[...])
    hb2_ref[...] = hb2
    t = jnp.maximum(_mm(hb2, w1[...], b1[...]), 0.0)
    he2 = _mm(t, w2[...], b2[...])
    msg2_ref[...] = _mm(he2 * ghn2, wm[...], bm[...])


def _node_final_body(h2_ref, cent2_ref, agg_ref, lng, lnb, wo, bo,
                     h3_ref):
    out = cent2_ref[...] + agg_ref[0] + agg_ref[1]
    out = _ln(out, lng[...], lnb[...])
    h3_ref[...] = h2_ref[...] + _mm(jnp.maximum(out, 0.0), wo[...], bo[...])


def _tc_call(body, grid, in_arrs, in_specs, out_shapes, out_specs):
    return pl.pallas_call(
        body,
        grid=(grid,),
        in_specs=in_specs,
        out_specs=out_specs,
        out_shape=out_shapes,
    )(*in_arrs)


def _agg_spec():
    return pl.BlockSpec((2, TN, 128), lambda i: (0, i, 0))


_gather_g12 = _make_geom_gather()
_gather_g3 = _make_gather((128, 128))
_gather_g4 = _make_gather((128, 256))
_scatter_s1 = _make_scatter1()
_scatter_s2 = _make_scatter2()


def _b2(v):
    return v.reshape(1, -1)


def _idx3(idx, ways, trips):
    pad = trips * ways * CH - N_EDGES
    return jnp.pad(idx, (0, pad)).reshape(trips, ways, CH)


def kernel(h_node, pos_node, h_bond, bond_index, batch, is_mol, is_frag, params):
    P = params
    nbe = P["nbe"][0]
    nbb = P["nbb"][0]
    bb = P["bb"][0]
    row = bond_index[0]
    col = bond_index[1]
    row32 = _idx3(row, NW, TRIPS32)
    col32 = _idx3(col, NW, TRIPS32)
    row16 = _idx3(row, NS, TRIPS16)
    col16 = _idx3(col, NS, TRIPS16)
    zeros_n = jnp.zeros((N_PAD, 128), jnp.float32)

    pos4 = jnp.zeros((N_NODES, 4), jnp.float32).at[:, :3].set(pos_node)
    pos4 = pos4.reshape(N_NODES * 4)
    off = np.zeros((1, 32), np.float32)
    off[0, :NUM_GAUSS] = np.linspace(0.0, CUTOFF, NUM_GAUSS)
    off = jnp.asarray(off)
    we_pad = jnp.zeros((32, 128), jnp.float32).at[:NUM_GAUSS].set(P["edge_emb"]["W"])

    ew = _full_spec

    h, hn_all, cent = _tc_call(
        _node_pre_body, N_NODES // TN,
        [h_node, P["lin_node"]["W"], _b2(P["lin_node"]["b"]),
         nbe["node_net"]["l1"]["W"], _b2(nbe["node_net"]["l1"]["b"]),
         nbe["node_net"]["l2"]["W"], _b2(nbe["node_net"]["l2"]["b"]),
         nbe["centroid"]["W"], _b2(nbe["centroid"]["b"])],
        [_row_spec(TN, 128)] + [ew((128, 128)), ew((1, 128))] * 4,
        [jax.ShapeDtypeStruct((N_NODES, 128), jnp.float32)] * 3,
        [_row_spec(TN, 128)] * 3,
    )

    ghn, d2 = _gather_g12(hn_all, pos4, row32, col32)
    d2 = d2.reshape(N_EDGES, 1)

    (msg,) = _tc_call(
        _edge_nbe_body, N_EDGES // TE,
        [d2, ghn, off, we_pad, _b2(P["edge_emb"]["b"]),
         nbe["edge_net"]["l1"]["W"], _b2(nbe["edge_net"]["l1"]["b"]),
         nbe["edge_net"]["l2"]["W"], _b2(nbe["edge_net"]["l2"]["b"]),
         nbe["msg_net"]["W"], _b2(nbe["msg_net"]["b"])],
        [_row_spec(TE, 1), _row_spec(TE, 128),
         ew((1, 32)), ew((32, 128)), ew((1, 128)),
         ew((128, 128)), ew((1, 128)), ew((128, 128)), ew((1, 128)),
         ew((128, 128)), ew((1, 128))],
        [jax.ShapeDtypeStruct((N_EDGES, 128), jnp.float32)],
        [_row_spec(TE, 128)],
    )

    aggr = _scatter_s1(msg, row32, zeros_n)

    h2, hn2, cent2 = _tc_call(
        _node_post_body, N_NODES // TN,
        [h, cent, aggr, _b2(nbe["ln_g"]), _b2(nbe["ln_b"]),
         nbe["out"]["W"], _b2(nbe["out"]["b"]),
         nbb["node_net"]["l1"]["W"], _b2(nbb["node_net"]["l1"]["b"]),
         nbb["node_net"]["l2"]["W"], _b2(nbb["node_net"]["l2"]["b"]),
         nbb["centroid"]["W"], _b2(nbb["centroid"]["b"])],
        [_row_spec(TN, 128)] * 2 + [_agg_spec()]
        + [ew((1, 128)), ew((1, 128))]
        + [ew((128, 128)), ew((1, 128))] * 4,
        [jax.ShapeDtypeStruct((N_NODES, 128), jnp.float32)] * 3,
        [_row_spec(TN, 128)] * 3,
    )

    gl, gr = _gather_g3(h2, row32, h2, col32)

    m_l_pre, m_r_pre, part = _tc_call(
        _edge_bond1_body, N_EDGES // TE,
        [h_bond, gl, gr,
         bb["ffn_l"]["bond_lin"]["W"], bb["ffn_l"]["node_lin"]["W"],
         bb["ffn_l"]["inter"]["l1"]["W"], _b2(bb["ffn_l"]["inter"]["l1"]["b"]),
         bb["ffn_l"]["inter"]["l2"]["W"], _b2(bb["ffn_l"]["inter"]["l2"]["b"]),
         bb["ffn_r"]["bond_lin"]["W"], bb["ffn_r"]["node_lin"]["W"],
         bb["ffn_r"]["inter"]["l1"]["W"], _b2(bb["ffn_r"]["inter"]["l1"]["b"]),
         bb["ffn_r"]["inter"]["l2"]["W"], _b2(bb["ffn_r"]["inter"]["l2"]["b"]),
         bb["node_ffn_l"]["W"], _b2(bb["node_ffn_l"]["b"]),
         bb["node_ffn_r"]["W"], _b2(bb["node_ffn_r"]["b"]),
         bb["self_ffn"]["W"], _b2(bb["self_ffn"]["b"])],
        [_row_spec(TE, 128)] * 3
        + [ew((128, 256)), ew((128, 256)), ew((256, 256)), ew((1, 256)),
           ew((256, 128)), ew((1, 128))] * 2
        + [ew((128, 128)), ew((1, 128))] * 3,
        [jax.ShapeDtypeStruct((N_EDGES, 128), jnp.float32)] * 3,
        [_row_spec(TE, 128)] * 3,
    )

    A12 = _scatter_s2(m_l_pre, col16, m_r_pre, row16, zeros_n)
    A1 = A12[0]
    tac = jnp.concatenate([A12[1][:N_NODES], hn2], axis=1)

    gA1, gac = _gather_g4(A1, row32, tac, col32)

    hb2, msg2 = _tc_call(
        _edge_bond2_body, N_EDGES // TE,
        [h_bond, gA1, gac, part,
         _b2(bb["ln_g"]), _b2(bb["ln_b"]),
         bb["out"]["W"], _b2(bb["out"]["b"]),
         nbb["edge_net"]["l1"]["W"], _b2(nbb["edge_net"]["l1"]["b"]),
         nbb["edge_net"]["l2"]["W"], _b2(nbb["edge_net"]["l2"]["b"]),
         nbb["msg_net"]["W"], _b2(nbb["msg_net"]["b"])],
        [_row_spec(TE, 128), _row_spec(TE, 128), _row_spec(TE, 256),
         _row_spec(TE, 128)]
        + [ew((1, 128)), ew((1, 128))]
        + [ew((128, 128)), ew((1, 128))] * 4,
        [jax.ShapeDtypeStruct((N_EDGES, 128), jnp.float32)] * 2,
        [_row_spec(TE, 128)] * 2,
    )

    aggr2 = _scatter_s1(msg2, row32, zeros_n)

    (h3,) = _tc_call(
        _node_final_body, N_NODES // TN,
        [h2, cent2, aggr2, _b2(nbb["ln_g"]), _b2(nbb["ln_b"]),
         nbb["out"]["W"], _b2(nbb["out"]["b"])],
        [_row_spec(TN, 128)] * 2 + [_agg_spec()]
        + [ew((1, 128)), ew((1, 128)), ew((128, 128)), ew((1, 128))],
        [jax.ShapeDtypeStruct((N_NODES, 128), jnp.float32)],
        [_row_spec(TN, 128)],
    )

    return h3, hb2


# speedup vs baseline: 3.3494x; 1.0429x over previous
import functools
import jax
import jax.numpy as jnp
import numpy as np
from jax import lax
from jax.experimental import pallas as pl
from jax.experimental.pallas import tpu as pltpu
from jax.experimental.pallas import tpu_sc as plsc

N_NODES = 10000
N_EDGES = 160000
NUM_GAUSS = 20
CUTOFF = 10.0

TE = 640
TN = 1000

NC, NS = 2, 16
NW = NC * NS
CH = 128
NCHUNK = N_EDGES // CH
TRIPS32 = -(-NCHUNK // NW)
TRIPS16 = -(-NCHUNK // NS)
N_PAD = 10240
NODE_SLICE = N_PAD // NS

_SC_MESH = plsc.VectorSubcoreMesh(core_axis_name="c", subcore_axis_name="s")


def _valid(wid, t, ways):
    return jnp.logical_and(t >= 0, wid + t * ways < NCHUNK)


def _wait(src, dst, sem):
    pltpu.make_async_copy(src, dst, sem).wait()


def _make_gather(dims):
    k = len(dims)
    scratch = []
    for d, dt in dims:
        scratch.append(pltpu.VMEM((TRIPS32, CH), jnp.int32))
        for b in range(2):
            scratch.append(pltpu.VMEM((CH, d), dt))
            scratch.append(pltpu.SemaphoreType.DMA)
            scratch.append(pltpu.SemaphoreType.DMA)
    out_type = [jax.ShapeDtypeStruct((N_EDGES, d), dt) for (d, dt) in dims]

    @functools.partial(pl.kernel, out_type=out_type, mesh=_SC_MESH,
                       scratch_types=scratch)
    def gk(*refs):
        tabs = refs[0:2 * k:2]
        idx3 = refs[1:2 * k:2]
        outs = refs[2 * k:3 * k]
        scr = refs[3 * k:]
        idx_all = [scr[7 * j] for j in range(k)]
        rows = [[scr[7 * j + 1 + 3 * b] for b in range(2)] for j in range(k)]
        semg = [[scr[7 * j + 2 + 3 * b] for b in range(2)] for j in range(k)]
        semw = [[scr[7 * j + 3 + 3 * b] for b in range(2)] for j in range(k)]
        wid = lax.axis_index("s") * NC + lax.axis_index("c")
        for j in range(k):
            pltpu.sync_copy(idx3[j].at[:, wid], idx_all[j])

        def trip(t, b):
            tm1, tm2 = t - 1, t - 2
            for j in range(k):
                @pl.when(_valid(wid, tm2, NW))
                def _(j=j):
                    _wait(rows[j][b], outs[j].at[pl.ds(0, CH)], semw[j][b])
            for j in range(k):
                @pl.when(_valid(wid, t, NW))
                def _(j=j):
                    pltpu.async_copy(tabs[j].at[idx_all[j].at[t]],
                                     rows[j][b], semg[j][b])
            for j in range(k):
                @pl.when(_valid(wid, tm1, NW))
                def _(j=j):
                    bp = 1 - b
                    _wait(tabs[j].at[pl.ds(0, CH)], rows[j][bp], semg[j][bp])
                    base = (wid + tm1 * NW) * CH
                    pltpu.async_copy(rows[j][bp], outs[j].at[pl.ds(base, CH)],
                                     semw[j][bp])

        def pair(i, carry):
            trip(2 * i, 0)
            trip(2 * i + 1, 1)
            return carry

        lax.fori_loop(0, (TRIPS32 + 3) // 2, pair, 0)

    return gk


def _make_geom_gather():
    scratch = [pltpu.VMEM((N_NODES * 4,), jnp.float32),
               pltpu.VMEM((TRIPS32, CH), jnp.int32),
               pltpu.VMEM((TRIPS32, CH), jnp.int32),
               pltpu.VMEM((CH, 128), jnp.float32),
               pltpu.VMEM((CH, 128), jnp.float32),
               pltpu.VMEM((CH,), jnp.float32),
               pltpu.SemaphoreType.DMA,
               pltpu.SemaphoreType.DMA,
               pltpu.SemaphoreType.DMA,
               pltpu.SemaphoreType.DMA]
    out_type = [jax.ShapeDtypeStruct((N_EDGES, 128), jnp.float32),
                jax.ShapeDtypeStruct((N_EDGES,), jnp.float32)]

    @functools.partial(pl.kernel, out_type=out_type, mesh=_SC_MESH,
                       scratch_types=scratch,
                       compiler_params=pltpu.CompilerParams(
                           needs_layout_passes=False))
    def gk(hn_tab, pos4, row3, col3, ghn_out, d2_out,
           pos_v, ridx_all, cidx_all, rows0, rows1, d2_v,
           semg0, semg1, semw0, semw1):
        rows = [rows0, rows1]
        semg = [semg0, semg1]
        semw = [semw0, semw1]
        wid = lax.axis_index("s") * NC + lax.axis_index("c")
        pltpu.sync_copy(pos4, pos_v)
        pltpu.sync_copy(row3.at[:, wid], ridx_all)
        pltpu.sync_copy(col3.at[:, wid], cidx_all)

        def trip(t, b):
            tm1, tm2 = t - 1, t - 2

            @pl.when(_valid(wid, tm2, NW))
            def _():
                _wait(rows[b], ghn_out.at[pl.ds(0, CH)], semw[b])

            @pl.when(_valid(wid, t, NW))
            def _():
                pltpu.async_copy(hn_tab.at[cidx_all.at[t]], rows[b], semg[b])
                for l in range(CH // 16):
                    ri = ridx_all[t, pl.ds(l * 16, 16)] * 4
                    ci = cidx_all[t, pl.ds(l * 16, 16)] * 4
                    dx = (plsc.load_gather(pos_v, [ri])
                          - plsc.load_gather(pos_v, [ci]))
                    dy = (plsc.load_gather(pos_v, [ri + 1])
                          - plsc.load_gather(pos_v, [ci + 1]))
                    dz = (plsc.load_gather(pos_v, [ri + 2])
                          - plsc.load_gather(pos_v, [ci + 2]))
                    d2_v[pl.ds(l * 16, 16)] = (dx * dx + dy * dy + dz * dz
                                               + 1e-8)
                pltpu.sync_copy(d2_v,
                                d2_out.at[pl.ds((wid + t * NW) * CH, CH)])

            @pl.when(_valid(wid, tm1, NW))
            def _():
                bp = 1 - b
                _wait(hn_tab.at[pl.ds(0, CH)], rows[bp], semg[bp])
                base = (wid + tm1 * NW) * CH
                pltpu.async_copy(rows[bp], ghn_out.at[pl.ds(base, CH)],
                                 semw[bp])

        def pair(i, carry):
            trip(2 * i, 0)
            trip(2 * i + 1, 1)
            return carry

        lax.fori_loop(0, (TRIPS32 + 3) // 2, pair, 0)

    return gk


def _scatter_loop(vals, idx_all, accum, rows, semv, wid, ways, trips):
    def fire(t, b):
        @pl.when(_valid(wid, t, ways))
        def _():
            base = (wid + t * ways) * CH
            pltpu.async_copy(vals.at[pl.ds(base, CH)], rows[b], semv[b])

    fire(0, 0)
    fire(1, 1)

    def trip(t, b):
        @pl.when(_valid(wid, t, ways))
        def _():
            _wait(vals.at[pl.ds(0, CH)], rows[b], semv[b])
            pltpu.sync_copy(rows[b], accum.at[idx_all.at[t]], add=True)
        fire(t + 2, b)

    def pair(i, carry):
        trip(2 * i, 0)
        trip(2 * i + 1, 1)
        return carry

    lax.fori_loop(0, (trips + 1) // 2, pair, 0)


def _make_scatter1():
    scratch = [pltpu.VMEM((TRIPS32, CH), jnp.int32),
               pltpu.VMEM((CH, 128), jnp.float32),
               pltpu.VMEM((CH, 128), jnp.float32),
               pltpu.SemaphoreType.DMA,
               pltpu.SemaphoreType.DMA,
               pltpu.VMEM_SHARED((N_PAD, 128), jnp.float32)]
    out_type = jax.ShapeDtypeStruct((NC, N_PAD, 128), jnp.float32)

    @functools.partial(pl.kernel, out_type=out_type, mesh=_SC_MESH,
                       scratch_types=scratch)
    def sk(vals, idx3, zeros, out, idx_all, rows0, rows1, semv0, semv1, accum):
        c = lax.axis_index("c")
        s = lax.axis_index("s")
        wid = s * NC + c
        base_n = s * NODE_SLICE
        pltpu.sync_copy(zeros.at[pl.ds(base_n, NODE_SLICE)],
                        accum.at[pl.ds(base_n, NODE_SLICE)])
        pltpu.sync_copy(idx3.at[:, wid], idx_all)
        plsc.subcore_barrier()
        _scatter_loop(vals, idx_all, accum, [rows0, rows1],
                      [semv0, semv1], wid, NW, TRIPS32)
        plsc.subcore_barrier()
        pltpu.sync_copy(accum.at[pl.ds(base_n, NODE_SLICE)],
                        out.at[c, pl.ds(base_n, NODE_SLICE)])

    return sk


def _make_scatter2():
    scratch = [pltpu.VMEM((TRIPS16, CH), jnp.int32),
               pltpu.VMEM((CH, 128), jnp.float32),
               pltpu.VMEM((CH, 128), jnp.float32),
               pltpu.SemaphoreType.DMA,
               pltpu.SemaphoreType.DMA,
               pltpu.VMEM_SHARED((N_PAD, 128), jnp.float32)]
    out_type = jax.ShapeDtypeStruct((NC, N_PAD, 128), jnp.float32)

    @functools.partial(pl.kernel, out_type=out_type, mesh=_SC_MESH,
                       scratch_types=scratch)
    def sk(vl, il3, vr, ir3, zeros, out, idx_all, rows0, rows1,
           semv0, semv1, accum):
        c = lax.axis_index("c")
        s = lax.axis_index("s")
        base_n = s * NODE_SLICE
        pltpu.sync_copy(zeros.at[pl.ds(base_n, NODE_SLICE)],
                        accum.at[pl.ds(base_n, NODE_SLICE)])

        def run(vals, idx3):
            pltpu.sync_copy(idx3.at[:, s], idx_all)
            plsc.subcore_barrier()
            _scatter_loop(vals, idx_all, accum, [rows0, rows1],
                          [semv0, semv1], s, NS, TRIPS16)
            plsc.subcore_barrier()
            pltpu.sync_copy(accum.at[pl.ds(base_n, NODE_SLICE)],
                            out.at[c, pl.ds(base_n, NODE_SLICE)])

        @pl.when(c == 0)
        def _():
            run(vl, il3)

        @pl.when(c == 1)
        def _():
            run(vr, ir3)

    return sk


def _full_spec(shape):
    return pl.BlockSpec(shape, lambda i: tuple(0 for _ in shape))


def _row_spec(block_rows, ncols):
    return pl.BlockSpec((block_rows, ncols), lambda i: (i, 0))


def _ln(x, g, b):
    m = jnp.mean(x, -1, keepdims=True)
    xc = x - m
    v = jnp.mean(xc * xc, -1, keepdims=True)
    return xc * jax.lax.rsqrt(v + 1e-5) * g + b


def _mm(x, w, b=None):
    y = jax.lax.dot_general(x.astype(jnp.bfloat16), w.astype(jnp.bfloat16),
                            (((1,), (0,)), ((), ())),
                            preferred_element_type=jnp.float32)
    if b is not None:
        y = y + b
    return y


def _node_pre_body(x_ref, wl, bl, w1, b1, w2, b2, wc, bc,
                   h_ref, hn_ref, cent_ref):
    x = x_ref[...]
    h = _mm(x, wl[...], bl[...])
    h_ref[...] = h
    t = jnp.maximum(_mm(h, w1[...], b1[...]), 0.0)
    hn_ref[...] = _mm(t, w2[...], b2[...])
    cent_ref[...] = _mm(h, wc[...], bc[...])


def _edge_nbe_body(d2_ref, ghn_ref, off, we, be, w1, b1, w2, b2,
                   wm, bm, msg_ref):
    ghn = ghn_ref[...]
    d = jnp.sqrt(d2_ref[...])
    step = CUTOFF / (NUM_GAUSS - 1)
    coeff = -0.5 / step ** 2
    diff = d - off[...]
    smear = jnp.exp(coeff * diff * diff)
    ea = _mm(smear, we[...], be[...])
    t = jnp.maximum(_mm(ea, w1[...], b1[...]), 0.0)
    he = _mm(t, w2[...], b2[...])
    msg_ref[...] = _mm(he * ghn, wm[...], bm[...])


def _node_post_body(h_ref, cent_ref, agg_ref, lng, lnb, wo, bo,
                    w1, b1, w2, b2, wc2, bc2,
                    h2_ref, hn2_ref, cent2_ref):
    out = cent_ref[...] + agg_ref[0] + agg_ref[1]
    out = _ln(out, lng[...], lnb[...])
    h2 = h_ref[...] + _mm(jnp.maximum(out, 0.0), wo[...], bo[...])
    h2_ref[...] = h2
    t = jnp.maximum(_mm(h2, w1[...], b1[...]), 0.0)
    hn2_ref[...] = _mm(t, w2[...], b2[...])
    cent2_ref[...] = _mm(h2, wc2[...], bc2[...])


def _edge_bond1_body(hb_ref, gl_ref, gr_ref,
                     wbl, wnl, w1l, b1l, w2l, b2l,
                     wbr, wnr, w1r, b1r, w2r, b2r,
                     wfl, bfl, wfr, bfr, ws, bs,
                     ml_ref, mr_ref, part_ref):
    hb = hb_ref[...]
    gl = gl_ref[...]
    gr = gr_ref[...]
    il = _mm(hb, wbl[...]) * _mm(gl, wnl[...])
    t = jnp.maximum(_mm(il, w1l[...], b1l[...]), 0.0)
    ml_ref[...] = _mm(t, w2l[...], b2l[...])
    ir = _mm(hb, wbr[...]) * _mm(gr, wnr[...])
    t = jnp.maximum(_mm(ir, w1r[...], b1r[...]), 0.0)
    mr_ref[...] = _mm(t, w2r[...], b2r[...])
    part_ref[...] = (_mm(gl, wfl[...], bfl[...]) + _mm(gr, wfr[...], bfr[...])
                     + _mm(hb, ws[...], bs[...])).astype(jnp.bfloat16)


def _edge_bond2_body(hb_ref, ga1_ref, gac_ref, part_ref,
                     lng, lnb, wo, bo, w1, b1, w2, b2, wm, bm,
                     hb2_ref, msg2_ref):
    gac = gac_ref[...]
    ga2 = jax.lax.bitcast_convert_type(gac << 16, jnp.float32)
    ghn2 = jax.lax.bitcast_convert_type(
        gac & jnp.int32(-65536), jnp.float32)
    pre = ga1_ref[...] + ga2 + part_ref[...].astype(jnp.float32)
    pre = _ln(pre, lng[...], lnb[...])
    hb2 = (hb_ref[...].astype(jnp.float32)
           + _mm(jnp.maximum(pre, 0.0), wo[...], bo[...]))
    hb2_ref[...] = hb2
    t = jnp.maximum(_mm(hb2, w1[...], b1[...]), 0.0)
    he2 = _mm(t, w2[...], b2[...])
    msg2_ref[...] = _mm(he2 * ghn2, wm[...], bm[...])


def _node_final_body(h2_ref, cent2_ref, agg_ref, lng, lnb, wo, bo,
                     h3_ref):
    out = cent2_ref[...] + agg_ref[0] + agg_ref[1]
    out = _ln(out, lng[...], lnb[...])
    h3_ref[...] = h2_ref[...] + _mm(jnp.maximum(out, 0.0), wo[...], bo[...])


def _tc_call(body, grid, in_arrs, in_specs, out_shapes, out_specs):
    return pl.pallas_call(
        body,
        grid=(grid,),
        in_specs=in_specs,
        out_specs=out_specs,
        out_shape=out_shapes,
    )(*in_arrs)


def _agg_spec():
    return pl.BlockSpec((2, TN, 128), lambda i: (0, i, 0))


_gather_g12 = _make_geom_gather()
_gather_g3 = _make_gather(((128, jnp.float32), (128, jnp.float32)))
_gather_g4 = _make_gather(((128, jnp.float32), (128, jnp.int32)))
_scatter_s1 = _make_scatter1()
_scatter_s2 = _make_scatter2()


def _b2(v):
    return v.reshape(1, -1)


def _idx3(idx, ways, trips):
    pad = trips * ways * CH - N_EDGES
    return jnp.pad(idx, (0, pad)).reshape(trips, ways, CH)


def kernel(h_node, pos_node, h_bond, bond_index, batch, is_mol, is_frag, params):
    P = params
    nbe = P["nbe"][0]
    nbb = P["nbb"][0]
    bb = P["bb"][0]
    row = bond_index[0]
    col = bond_index[1]
    row32 = _idx3(row, NW, TRIPS32)
    col32 = _idx3(col, NW, TRIPS32)
    row16 = _idx3(row, NS, TRIPS16)
    col16 = _idx3(col, NS, TRIPS16)
    zeros_n = jnp.zeros((N_PAD, 128), jnp.float32)

    pos4 = jnp.zeros((N_NODES, 4), jnp.float32).at[:, :3].set(pos_node)
    pos4 = pos4.reshape(N_NODES * 4)
    off = np.zeros((1, 32), np.float32)
    off[0, :NUM_GAUSS] = np.linspace(0.0, CUTOFF, NUM_GAUSS)
    off = jnp.asarray(off)
    we_pad = jnp.zeros((32, 128), jnp.float32).at[:NUM_GAUSS].set(P["edge_emb"]["W"])

    ew = _full_spec

    h, hn_all, cent = _tc_call(
        _node_pre_body, N_NODES // TN,
        [h_node, P["lin_node"]["W"], _b2(P["lin_node"]["b"]),
         nbe["node_net"]["l1"]["W"], _b2(nbe["node_net"]["l1"]["b"]),
         nbe["node_net"]["l2"]["W"], _b2(nbe["node_net"]["l2"]["b"]),
         nbe["centroid"]["W"], _b2(nbe["centroid"]["b"])],
        [_row_spec(TN, 128)] + [ew((128, 128)), ew((1, 128))] * 4,
        [jax.ShapeDtypeStruct((N_NODES, 128), jnp.float32)] * 3,
        [_row_spec(TN, 128)] * 3,
    )

    ghn, d2 = _gather_g12(hn_all, pos4, row32, col32)
    d2 = d2.reshape(N_EDGES, 1)

    (msg,) = _tc_call(
        _edge_nbe_body, N_EDGES // TE,
        [d2, ghn, off, we_pad, _b2(P["edge_emb"]["b"]),
         nbe["edge_net"]["l1"]["W"], _b2(nbe["edge_net"]["l1"]["b"]),
         nbe["edge_net"]["l2"]["W"], _b2(nbe["edge_net"]["l2"]["b"]),
         nbe["msg_net"]["W"], _b2(nbe["msg_net"]["b"])],
        [_row_spec(TE, 1), _row_spec(TE, 128),
         ew((1, 32)), ew((32, 128)), ew((1, 128)),
         ew((128, 128)), ew((1, 128)), ew((128, 128)), ew((1, 128)),
         ew((128, 128)), ew((1, 128))],
        [jax.ShapeDtypeStruct((N_EDGES, 128), jnp.float32)],
        [_row_spec(TE, 128)],
    )

    aggr = _scatter_s1(msg, row32, zeros_n)

    h2, hn2, cent2 = _tc_call(
        _node_post_body, N_NODES // TN,
        [h, cent, aggr, _b2(nbe["ln_g"]), _b2(nbe["ln_b"]),
         nbe["out"]["W"], _b2(nbe["out"]["b"]),
         nbb["node_net"]["l1"]["W"], _b2(nbb["node_net"]["l1"]["b"]),
         nbb["node_net"]["l2"]["W"], _b2(nbb["node_net"]["l2"]["b"]),
         nbb["centroid"]["W"], _b2(nbb["centroid"]["b"])],
        [_row_spec(TN, 128)] * 2 + [_agg_spec()]
        + [ew((1, 128)), ew((1, 128))]
        + [ew((128, 128)), ew((1, 128))] * 4,
        [jax.ShapeDtypeStruct((N_NODES, 128), jnp.float32)] * 3,
        [_row_spec(TN, 128)] * 3,
    )

    gl, gr = _gather_g3(h2, row32, h2, col32)

    hb16 = h_bond.astype(jnp.bfloat16)
    m_l_pre, m_r_pre, part = _tc_call(
        _edge_bond1_body, N_EDGES // TE,
        [hb16, gl, gr,
         bb["ffn_l"]["bond_lin"]["W"], bb["ffn_l"]["node_lin"]["W"],
         bb["ffn_l"]["inter"]["l1"]["W"], _b2(bb["ffn_l"]["inter"]["l1"]["b"]),
         bb["ffn_l"]["inter"]["l2"]["W"], _b2(bb["ffn_l"]["inter"]["l2"]["b"]),
         bb["ffn_r"]["bond_lin"]["W"], bb["ffn_r"]["node_lin"]["W"],
         bb["ffn_r"]["inter"]["l1"]["W"], _b2(bb["ffn_r"]["inter"]["l1"]["b"]),
         bb["ffn_r"]["inter"]["l2"]["W"], _b2(bb["ffn_r"]["inter"]["l2"]["b"]),
         bb["node_ffn_l"]["W"], _b2(bb["node_ffn_l"]["b"]),
         bb["node_ffn_r"]["W"], _b2(bb["node_ffn_r"]["b"]),
         bb["self_ffn"]["W"], _b2(bb["self_ffn"]["b"])],
        [_row_spec(TE, 128)] * 3
        + [ew((128, 256)), ew((128, 256)), ew((256, 256)), ew((1, 256)),
           ew((256, 128)), ew((1, 128))] * 2
        + [ew((128, 128)), ew((1, 128))] * 3,
        [jax.ShapeDtypeStruct((N_EDGES, 128), jnp.float32)] * 2
        + [jax.ShapeDtypeStruct((N_EDGES, 128), jnp.bfloat16)],
        [_row_spec(TE, 128)] * 3,
    )

    A12 = _scatter_s2(m_l_pre, col16, m_r_pre, row16, zeros_n)
    A1 = A12[0]
    a2b = jax.lax.bitcast_convert_type(A12[1][:N_NODES], jnp.uint32)
    hnb = jax.lax.bitcast_convert_type(hn2, jnp.uint32)
    tac = jax.lax.bitcast_convert_type(
        (hnb & jnp.uint32(0xFFFF0000)) | (a2b >> 16), jnp.int32)

    gA1, gac = _gather_g4(A1, row32, tac, col32)

    hb2, msg2 = _tc_call(
        _edge_bond2_body, N_EDGES // TE,
        [hb16, gA1, gac, part,
         _b2(bb["ln_g"]), _b2(bb["ln_b"]),
         bb["out"]["W"], _b2(bb["out"]["b"]),
         nbb["edge_net"]["l1"]["W"], _b2(nbb["edge_net"]["l1"]["b"]),
         nbb["edge_net"]["l2"]["W"], _b2(nbb["edge_net"]["l2"]["b"]),
         nbb["msg_net"]["W"], _b2(nbb["msg_net"]["b"])],
        [_row_spec(TE, 128)] * 4
        + [ew((1, 128)), ew((1, 128))]
        + [ew((128, 128)), ew((1, 128))] * 4,
        [jax.ShapeDtypeStruct((N_EDGES, 128), jnp.float32)] * 2,
        [_row_spec(TE, 128)] * 2,
    )

    aggr2 = _scatter_s1(msg2, row32, zeros_n)

    (h3,) = _tc_call(
        _node_final_body, N_NODES // TN,
        [h2, cent2, aggr2, _b2(nbb["ln_g"]), _b2(nbb["ln_b"]),
         nbb["out"]["W"], _b2(nbb["out"]["b"])],
        [_row_spec(TN, 128)] * 2 + [_agg_spec()]
        + [ew((1, 128)), ew((1, 128)), ew((128, 128)), ew((1, 128))],
        [jax.ShapeDtypeStruct((N_NODES, 128), jnp.float32)],
        [_row_spec(TN, 128)],
    )

    return h3, hb2


# speedup vs baseline: 4.4470x; 1.3277x over previous
import functools
import jax
import jax.numpy as jnp
import numpy as np
from jax import lax
from jax.experimental import pallas as pl
from jax.experimental.pallas import tpu as pltpu
from jax.experimental.pallas import tpu_sc as plsc

N_NODES = 10000
N_EDGES = 160000
NUM_GAUSS = 20
CUTOFF = 10.0

TE = 3200
TN = 1000

NC, NS = 2, 16
NW = NC * NS
CH = 128
NCHUNK = N_EDGES // CH
TRIPS32 = -(-NCHUNK // NW)
TRIPS16 = -(-NCHUNK // NS)
N_PAD = 10240
NODE_SLICE = N_PAD // NS

_SC_MESH = plsc.VectorSubcoreMesh(core_axis_name="c", subcore_axis_name="s")


def _valid(wid, t, ways):
    return jnp.logical_and(t >= 0, wid + t * ways < NCHUNK)


def _wait(src, dst, sem):
    pltpu.make_async_copy(src, dst, sem).wait()


def _make_gather(dims):
    k = len(dims)
    scratch = []
    for d, dt in dims:
        scratch.append(pltpu.VMEM((TRIPS32, CH), jnp.int32))
        for b in range(2):
            scratch.append(pltpu.VMEM((CH, d), dt))
            scratch.append(pltpu.SemaphoreType.DMA)
            scratch.append(pltpu.SemaphoreType.DMA)
    out_type = [jax.ShapeDtypeStruct((N_EDGES, d), dt) for (d, dt) in dims]

    @functools.partial(pl.kernel, out_type=out_type, mesh=_SC_MESH,
                       scratch_types=scratch)
    def gk(*refs):
        tabs = refs[0:2 * k:2]
        idx3 = refs[1:2 * k:2]
        outs = refs[2 * k:3 * k]
        scr = refs[3 * k:]
        idx_all = [scr[7 * j] for j in range(k)]
        rows = [[scr[7 * j + 1 + 3 * b] for b in range(2)] for j in range(k)]
        semg = [[scr[7 * j + 2 + 3 * b] for b in range(2)] for j in range(k)]
        semw = [[scr[7 * j + 3 + 3 * b] for b in range(2)] for j in range(k)]
        wid = lax.axis_index("s") * NC + lax.axis_index("c")
        for j in range(k):
            pltpu.sync_copy(idx3[j].at[:, wid], idx_all[j])

        def trip(t, b):
            tm1, tm2 = t - 1, t - 2
            for j in range(k):
                @pl.when(_valid(wid, tm2, NW))
                def _(j=j):
                    _wait(rows[j][b], outs[j].at[pl.ds(0, CH)], semw[j][b])
            for j in range(k):
                @pl.when(_valid(wid, t, NW))
                def _(j=j):
                    pltpu.async_copy(tabs[j].at[idx_all[j].at[t]],
                                     rows[j][b], semg[j][b])
            for j in range(k):
                @pl.when(_valid(wid, tm1, NW))
                def _(j=j):
                    bp = 1 - b
                    _wait(tabs[j].at[pl.ds(0, CH)], rows[j][bp], semg[j][bp])
                    base = (wid + tm1 * NW) * CH
                    pltpu.async_copy(rows[j][bp], outs[j].at[pl.ds(base, CH)],
                                     semw[j][bp])

        def pair(i, carry):
            trip(2 * i, 0)
            trip(2 * i + 1, 1)
            return carry

        lax.fori_loop(0, (TRIPS32 + 3) // 2, pair, 0)

    return gk


def _make_geom_gather():
    scratch = [pltpu.VMEM((N_NODES * 4,), jnp.float32),
               pltpu.VMEM((TRIPS32, CH), jnp.int32),
               pltpu.VMEM((TRIPS32, CH), jnp.int32),
               pltpu.VMEM((CH, 128), jnp.float32),
               pltpu.VMEM((CH, 128), jnp.float32),
               pltpu.VMEM((CH,), jnp.float32),
               pltpu.SemaphoreType.DMA,
               pltpu.SemaphoreType.DMA,
               pltpu.SemaphoreType.DMA,
               pltpu.SemaphoreType.DMA]
    out_type = [jax.ShapeDtypeStruct((N_EDGES, 128), jnp.float32),
                jax.ShapeDtypeStruct((N_EDGES,), jnp.float32)]

    @functools.partial(pl.kernel, out_type=out_type, mesh=_SC_MESH,
                       scratch_types=scratch,
                       compiler_params=pltpu.CompilerParams(
                           needs_layout_passes=False))
    def gk(hn_tab, pos4, row3, col3, ghn_out, d2_out,
           pos_v, ridx_all, cidx_all, rows0, rows1, d2_v,
           semg0, semg1, semw0, semw1):
        rows = [rows0, rows1]
        semg = [semg0, semg1]
        semw = [semw0, semw1]
        wid = lax.axis_index("s") * NC + lax.axis_index("c")
        pltpu.sync_copy(pos4, pos_v)
        pltpu.sync_copy(row3.at[:, wid], ridx_all)
        pltpu.sync_copy(col3.at[:, wid], cidx_all)

        def trip(t, b):
            tm1, tm2 = t - 1, t - 2

            @pl.when(_valid(wid, tm2, NW))
            def _():
                _wait(rows[b], ghn_out.at[pl.ds(0, CH)], semw[b])

            @pl.when(_valid(wid, t, NW))
            def _():
                pltpu.async_copy(hn_tab.at[cidx_all.at[t]], rows[b], semg[b])
                for l in range(CH // 16):
                    ri = ridx_all[t, pl.ds(l * 16, 16)] * 4
                    ci = cidx_all[t, pl.ds(l * 16, 16)] * 4
                    dx = (plsc.load_gather(pos_v, [ri])
                          - plsc.load_gather(pos_v, [ci]))
                    dy = (plsc.load_gather(pos_v, [ri + 1])
                          - plsc.load_gather(pos_v, [ci + 1]))
                    dz = (plsc.load_gather(pos_v, [ri + 2])
                          - plsc.load_gather(pos_v, [ci + 2]))
                    d2_v[pl.ds(l * 16, 16)] = (dx * dx + dy * dy + dz * dz
                                               + 1e-8)
                pltpu.sync_copy(d2_v,
                                d2_out.at[pl.ds((wid + t * NW) * CH, CH)])

            @pl.when(_valid(wid, tm1, NW))
            def _():
                bp = 1 - b
                _wait(hn_tab.at[pl.ds(0, CH)], rows[bp], semg[bp])
                base = (wid + tm1 * NW) * CH
                pltpu.async_copy(rows[bp], ghn_out.at[pl.ds(base, CH)],
                                 semw[bp])

        def pair(i, carry):
            trip(2 * i, 0)
            trip(2 * i + 1, 1)
            return carry

        lax.fori_loop(0, (TRIPS32 + 3) // 2, pair, 0)

    return gk


def _scatter_loop(vals, idx_all, accum, rows, semv, wid, ways, trips):
    def fire(t, b):
        @pl.when(_valid(wid, t, ways))
        def _():
            base = (wid + t * ways) * CH
            pltpu.async_copy(vals.at[pl.ds(base, CH)], rows[b], semv[b])

    fire(0, 0)
    fire(1, 1)

    def trip(t, b):
        @pl.when(_valid(wid, t, ways))
        def _():
            _wait(vals.at[pl.ds(0, CH)], rows[b], semv[b])
            pltpu.sync_copy(rows[b], accum.at[idx_all.at[t]], add=True)
        fire(t + 2, b)

    def pair(i, carry):
        trip(2 * i, 0)
        trip(2 * i + 1, 1)
        return carry

    lax.fori_loop(0, (trips + 1) // 2, pair, 0)


def _make_scatter1():
    scratch = [pltpu.VMEM((TRIPS32, CH), jnp.int32),
               pltpu.VMEM((CH, 128), jnp.float32),
               pltpu.VMEM((CH, 128), jnp.float32),
               pltpu.SemaphoreType.DMA,
               pltpu.SemaphoreType.DMA,
               pltpu.VMEM_SHARED((N_PAD, 128), jnp.float32)]
    out_type = jax.ShapeDtypeStruct((NC, N_PAD, 128), jnp.float32)

    @functools.partial(pl.kernel, out_type=out_type, mesh=_SC_MESH,
                       scratch_types=scratch)
    def sk(vals, idx3, zeros, out, idx_all, rows0, rows1, semv0, semv1, accum):
        c = lax.axis_index("c")
        s = lax.axis_index("s")
        wid = s * NC + c
        base_n = s * NODE_SLICE
        pltpu.sync_copy(zeros.at[pl.ds(base_n, NODE_SLICE)],
                        accum.at[pl.ds(base_n, NODE_SLICE)])
        pltpu.sync_copy(idx3.at[:, wid], idx_all)
        plsc.subcore_barrier()
        _scatter_loop(vals, idx_all, accum, [rows0, rows1],
                      [semv0, semv1], wid, NW, TRIPS32)
        plsc.subcore_barrier()
        pltpu.sync_copy(accum.at[pl.ds(base_n, NODE_SLICE)],
                        out.at[c, pl.ds(base_n, NODE_SLICE)])

    return sk


def _make_scatter2():
    scratch = [pltpu.VMEM((TRIPS16, CH), jnp.int32),
               pltpu.VMEM((CH, 128), jnp.float32),
               pltpu.VMEM((CH, 128), jnp.float32),
               pltpu.SemaphoreType.DMA,
               pltpu.SemaphoreType.DMA,
               pltpu.VMEM_SHARED((N_PAD, 128), jnp.float32)]
    out_type = jax.ShapeDtypeStruct((NC, N_PAD, 128), jnp.float32)

    @functools.partial(pl.kernel, out_type=out_type, mesh=_SC_MESH,
                       scratch_types=scratch)
    def sk(vl, il3, vr, ir3, zeros, out, idx_all, rows0, rows1,
           semv0, semv1, accum):
        c = lax.axis_index("c")
        s = lax.axis_index("s")
        base_n = s * NODE_SLICE
        pltpu.sync_copy(zeros.at[pl.ds(base_n, NODE_SLICE)],
                        accum.at[pl.ds(base_n, NODE_SLICE)])

        def run(vals, idx3):
            pltpu.sync_copy(idx3.at[:, s], idx_all)
            plsc.subcore_barrier()
            _scatter_loop(vals, idx_all, accum, [rows0, rows1],
                          [semv0, semv1], s, NS, TRIPS16)
            plsc.subcore_barrier()
            pltpu.sync_copy(accum.at[pl.ds(base_n, NODE_SLICE)],
                            out.at[c, pl.ds(base_n, NODE_SLICE)])

        @pl.when(c == 0)
        def _():
            run(vl, il3)

        @pl.when(c == 1)
        def _():
            run(vr, ir3)

    return sk


def _full_spec(shape):
    return pl.BlockSpec(shape, lambda i: tuple(0 for _ in shape))


def _row_spec(block_rows, ncols):
    return pl.BlockSpec((block_rows, ncols), lambda i: (i, 0))


def _ln(x, g, b):
    m = jnp.mean(x, -1, keepdims=True)
    xc = x - m
    v = jnp.mean(xc * xc, -1, keepdims=True)
    return xc * jax.lax.rsqrt(v + 1e-5) * g + b


def _mm(x, w, b=None):
    y = jax.lax.dot_general(x.astype(jnp.bfloat16), w.astype(jnp.bfloat16),
                            (((1,), (0,)), ((), ())),
                            preferred_element_type=jnp.float32)
    if b is not None:
        y = y + b
    return y


def _node_pre_body(x_ref, wl, bl, w1, b1, w2, b2, wc, bc,
                   h_ref, hn_ref, cent_ref):
    x = x_ref[...]
    h = _mm(x, wl[...], bl[...])
    h_ref[...] = h
    t = jnp.maximum(_mm(h, w1[...], b1[...]), 0.0)
    hn_ref[...] = _mm(t, w2[...], b2[...])
    cent_ref[...] = _mm(h, wc[...], bc[...])


def _edge_nbe_body(d2_ref, ghn_ref, off, we, be, w1, b1, w2, b2,
                   wm, bm, msg_ref):
    ghn = ghn_ref[...]
    d = jnp.sqrt(d2_ref[...])
    step = CUTOFF / (NUM_GAUSS - 1)
    coeff = -0.5 / step ** 2
    diff = d - off[...]
    smear = jnp.exp(coeff * diff * diff)
    ea = _mm(smear, we[...], be[...])
    t = jnp.maximum(_mm(ea, w1[...], b1[...]), 0.0)
    he = _mm(t, w2[...], b2[...])
    msg_ref[...] = _mm(he * ghn, wm[...], bm[...])


def _node_post_body(h_ref, cent_ref, agg_ref, lng, lnb, wo, bo,
                    w1, b1, w2, b2, wc2, bc2,
                    h2_ref, hn2_ref, cent2_ref):
    out = cent_ref[...] + agg_ref[0] + agg_ref[1]
    out = _ln(out, lng[...], lnb[...])
    h2 = h_ref[...] + _mm(jnp.maximum(out, 0.0), wo[...], bo[...])
    h2_ref[...] = h2
    t = jnp.maximum(_mm(h2, w1[...], b1[...]), 0.0)
    hn2_ref[...] = _mm(t, w2[...], b2[...])
    cent2_ref[...] = _mm(h2, wc2[...], bc2[...])


def _edge_bond1_body(hb_ref, gl_ref, gr_ref,
                     wbl, wnl, w1l, b1l, w2l, b2l,
                     wbr, wnr, w1r, b1r, w2r, b2r,
                     wfl, bfl, wfr, bfr, ws, bs,
                     ml_ref, mr_ref, part_ref):
    hb = hb_ref[...]
    gl = gl_ref[...]
    gr = gr_ref[...]
    il = _mm(hb, wbl[...]) * _mm(gl, wnl[...])
    t = jnp.maximum(_mm(il, w1l[...], b1l[...]), 0.0)
    ml_ref[...] = _mm(t, w2l[...], b2l[...])
    ir = _mm(hb, wbr[...]) * _mm(gr, wnr[...])
    t = jnp.maximum(_mm(ir, w1r[...], b1r[...]), 0.0)
    mr_ref[...] = _mm(t, w2r[...], b2r[...])
    part_ref[...] = (_mm(gl, wfl[...], bfl[...]) + _mm(gr, wfr[...], bfr[...])
                     + _mm(hb, ws[...], bs[...])).astype(jnp.bfloat16)


def _edge_bond2_body(hb_ref, ga1_ref, gac_ref, part_ref,
                     lng, lnb, wo, bo, w1, b1, w2, b2, wm, bm,
                     hb2_ref, msg2_ref):
    gac = gac_ref[...]
    ga2 = jax.lax.bitcast_convert_type(gac << 16, jnp.float32)
    ghn2 = jax.lax.bitcast_convert_type(
        gac & jnp.int32(-65536), jnp.float32)
    pre = ga1_ref[...] + ga2 + part_ref[...].astype(jnp.float32)
    pre = _ln(pre, lng[...], lnb[...])
    hb2 = (hb_ref[...].astype(jnp.float32)
           + _mm(jnp.maximum(pre, 0.0), wo[...], bo[...]))
    hb2_ref[...] = hb2
    t = jnp.maximum(_mm(hb2, w1[...], b1[...]), 0.0)
    he2 = _mm(t, w2[...], b2[...])
    msg2_ref[...] = _mm(he2 * ghn2, wm[...], bm[...])


def _node_final_body(h2_ref, cent2_ref, agg_ref, lng, lnb, wo, bo,
                     h3_ref):
    out = cent2_ref[...] + agg_ref[0] + agg_ref[1]
    out = _ln(out, lng[...], lnb[...])
    h3_ref[...] = h2_ref[...] + _mm(jnp.maximum(out, 0.0), wo[...], bo[...])


def _tc_call(body, grid, in_arrs, in_specs, out_shapes, out_specs):
    return pl.pallas_call(
        body,
        grid=(grid,),
        in_specs=in_specs,
        out_specs=out_specs,
        out_shape=out_shapes,
    )(*in_arrs)


def _agg_spec():
    return pl.BlockSpec((2, TN, 128), lambda i: (0, i, 0))


_gather_g12 = _make_geom_gather()
_gather_g3 = _make_gather(((128, jnp.float32), (128, jnp.float32)))
_gather_g4 = _make_gather(((128, jnp.float32), (128, jnp.int32)))
_scatter_s1 = _make_scatter1()
_scatter_s2 = _make_scatter2()


def _b2(v):
    return v.reshape(1, -1)


def _idx3(idx, ways, trips):
    pad = trips * ways * CH - N_EDGES
    return jnp.pad(idx, (0, pad)).reshape(trips, ways, CH)


def kernel(h_node, pos_node, h_bond, bond_index, batch, is_mol, is_frag, params):
    P = params
    nbe = P["nbe"][0]
    nbb = P["nbb"][0]
    bb = P["bb"][0]
    row = bond_index[0]
    col = bond_index[1]
    row32 = _idx3(row, NW, TRIPS32)
    col32 = _idx3(col, NW, TRIPS32)
    row16 = _idx3(row, NS, TRIPS16)
    col16 = _idx3(col, NS, TRIPS16)
    zeros_n = jnp.zeros((N_PAD, 128), jnp.float32)

    pos4 = jnp.zeros((N_NODES, 4), jnp.float32).at[:, :3].set(pos_node)
    pos4 = pos4.reshape(N_NODES * 4)
    off = np.zeros((1, 32), np.float32)
    off[0, :NUM_GAUSS] = np.linspace(0.0, CUTOFF, NUM_GAUSS)
    off = jnp.asarray(off)
    we_pad = jnp.zeros((32, 128), jnp.float32).at[:NUM_GAUSS].set(P["edge_emb"]["W"])

    ew = _full_spec

    h, hn_all, cent = _tc_call(
        _node_pre_body, N_NODES // TN,
        [h_node, P["lin_node"]["W"], _b2(P["lin_node"]["b"]),
         nbe["node_net"]["l1"]["W"], _b2(nbe["node_net"]["l1"]["b"]),
         nbe["node_net"]["l2"]["W"], _b2(nbe["node_net"]["l2"]["b"]),
         nbe["centroid"]["W"], _b2(nbe["centroid"]["b"])],
        [_row_spec(TN, 128)] + [ew((128, 128)), ew((1, 128))] * 4,
        [jax.ShapeDtypeStruct((N_NODES, 128), jnp.float32)] * 3,
        [_row_spec(TN, 128)] * 3,
    )

    ghn, d2 = _gather_g12(hn_all, pos4, row32, col32)
    d2 = d2.reshape(N_EDGES, 1)

    (msg,) = _tc_call(
        _edge_nbe_body, N_EDGES // TE,
        [d2, ghn, off, we_pad, _b2(P["edge_emb"]["b"]),
         nbe["edge_net"]["l1"]["W"], _b2(nbe["edge_net"]["l1"]["b"]),
         nbe["edge_net"]["l2"]["W"], _b2(nbe["edge_net"]["l2"]["b"]),
         nbe["msg_net"]["W"], _b2(nbe["msg_net"]["b"])],
        [_row_spec(TE, 1), _row_spec(TE, 128),
         ew((1, 32)), ew((32, 128)), ew((1, 128)),
         ew((128, 128)), ew((1, 128)), ew((128, 128)), ew((1, 128)),
         ew((128, 128)), ew((1, 128))],
        [jax.ShapeDtypeStruct((N_EDGES, 128), jnp.float32)],
        [_row_spec(TE, 128)],
    )

    aggr = _scatter_s1(msg, row32, zeros_n)

    h2, hn2, cent2 = _tc_call(
        _node_post_body, N_NODES // TN,
        [h, cent, aggr, _b2(nbe["ln_g"]), _b2(nbe["ln_b"]),
         nbe["out"]["W"], _b2(nbe["out"]["b"]),
         nbb["node_net"]["l1"]["W"], _b2(nbb["node_net"]["l1"]["b"]),
         nbb["node_net"]["l2"]["W"], _b2(nbb["node_net"]["l2"]["b"]),
         nbb["centroid"]["W"], _b2(nbb["centroid"]["b"])],
        [_row_spec(TN, 128)] * 2 + [_agg_spec()]
        + [ew((1, 128)), ew((1, 128))]
        + [ew((128, 128)), ew((1, 128))] * 4,
        [jax.ShapeDtypeStruct((N_NODES, 128), jnp.float32)] * 3,
        [_row_spec(TN, 128)] * 3,
    )

    gl, gr = _gather_g3(h2, row32, h2, col32)

    hb16 = h_bond.astype(jnp.bfloat16)
    m_l_pre, m_r_pre, part = _tc_call(
        _edge_bond1_body, N_EDGES // TE,
        [hb16, gl, gr,
         bb["ffn_l"]["bond_lin"]["W"], bb["ffn_l"]["node_lin"]["W"],
         bb["ffn_l"]["inter"]["l1"]["W"], _b2(bb["ffn_l"]["inter"]["l1"]["b"]),
         bb["ffn_l"]["inter"]["l2"]["W"], _b2(bb["ffn_l"]["inter"]["l2"]["b"]),
         bb["ffn_r"]["bond_lin"]["W"], bb["ffn_r"]["node_lin"]["W"],
         bb["ffn_r"]["inter"]["l1"]["W"], _b2(bb["ffn_r"]["inter"]["l1"]["b"]),
         bb["ffn_r"]["inter"]["l2"]["W"], _b2(bb["ffn_r"]["inter"]["l2"]["b"]),
         bb["node_ffn_l"]["W"], _b2(bb["node_ffn_l"]["b"]),
         bb["node_ffn_r"]["W"], _b2(bb["node_ffn_r"]["b"]),
         bb["self_ffn"]["W"], _b2(bb["self_ffn"]["b"])],
        [_row_spec(TE, 128)] * 3
        + [ew((128, 256)), ew((128, 256)), ew((256, 256)), ew((1, 256)),
           ew((256, 128)), ew((1, 128))] * 2
        + [ew((128, 128)), ew((1, 128))] * 3,
        [jax.ShapeDtypeStruct((N_EDGES, 128), jnp.float32)] * 2
        + [jax.ShapeDtypeStruct((N_EDGES, 128), jnp.bfloat16)],
        [_row_spec(TE, 128)] * 3,
    )

    A12 = _scatter_s2(m_l_pre, col16, m_r_pre, row16, zeros_n)
    A1 = A12[0]
    a2b = jax.lax.bitcast_convert_type(A12[1][:N_NODES], jnp.uint32)
    hnb = jax.lax.bitcast_convert_type(hn2, jnp.uint32)
    tac = jax.lax.bitcast_convert_type(
        (hnb & jnp.uint32(0xFFFF0000)) | (a2b >> 16), jnp.int32)

    gA1, gac = _gather_g4(A1, row32, tac, col32)

    hb2, msg2 = _tc_call(
        _edge_bond2_body, N_EDGES // TE,
        [hb16, gA1, gac, part,
         _b2(bb["ln_g"]), _b2(bb["ln_b"]),
         bb["out"]["W"], _b2(bb["out"]["b"]),
         nbb["edge_net"]["l1"]["W"], _b2(nbb["edge_net"]["l1"]["b"]),
         nbb["edge_net"]["l2"]["W"], _b2(nbb["edge_net"]["l2"]["b"]),
         nbb["msg_net"]["W"], _b2(nbb["msg_net"]["b"])],
        [_row_spec(TE, 128)] * 4
        + [ew((1, 128)), ew((1, 128))]
        + [ew((128, 128)), ew((1, 128))] * 4,
        [jax.ShapeDtypeStruct((N_EDGES, 128), jnp.float32)] * 2,
        [_row_spec(TE, 128)] * 2,
    )

    aggr2 = _scatter_s1(msg2, row32, zeros_n)

    (h3,) = _tc_call(
        _node_final_body, N_NODES // TN,
        [h2, cent2, aggr2, _b2(nbb["ln_g"]), _b2(nbb["ln_b"]),
         nbb["out"]["W"], _b2(nbb["out"]["b"])],
        [_row_spec(TN, 128)] * 2 + [_agg_spec()]
        + [ew((1, 128)), ew((1, 128)), ew((128, 128)), ew((1, 128))],
        [jax.ShapeDtypeStruct((N_NODES, 128), jnp.float32)],
        [_row_spec(TN, 128)],
    )

    return h3, hb2


# speedup vs baseline: 4.5518x; 1.0236x over previous
import functools
import jax
import jax.numpy as jnp
import numpy as np
from jax import lax
from jax.experimental import pallas as pl
from jax.experimental.pallas import tpu as pltpu
from jax.experimental.pallas import tpu_sc as plsc

N_NODES = 10000
N_EDGES = 160000
NUM_GAUSS = 20
CUTOFF = 10.0

TE = 6400
TN = 1000

NC, NS = 2, 16
NW = NC * NS
CH = 128
NCHUNK = N_EDGES // CH
TRIPS32 = -(-NCHUNK // NW)
TRIPS16 = -(-NCHUNK // NS)
N_PAD = 10240
NODE_SLICE = N_PAD // NS

_SC_MESH = plsc.VectorSubcoreMesh(core_axis_name="c", subcore_axis_name="s")


def _valid(wid, t, ways):
    return jnp.logical_and(t >= 0, wid + t * ways < NCHUNK)


def _wait(src, dst, sem):
    pltpu.make_async_copy(src, dst, sem).wait()


def _make_gather(dims):
    k = len(dims)
    scratch = []
    for d, dt in dims:
        scratch.append(pltpu.VMEM((TRIPS32, CH), jnp.int32))
        for b in range(2):
            scratch.append(pltpu.VMEM((CH, d), dt))
            scratch.append(pltpu.SemaphoreType.DMA)
            scratch.append(pltpu.SemaphoreType.DMA)
    out_type = [jax.ShapeDtypeStruct((N_EDGES, d), dt) for (d, dt) in dims]

    @functools.partial(pl.kernel, out_type=out_type, mesh=_SC_MESH,
                       scratch_types=scratch)
    def gk(*refs):
        tabs = refs[0:2 * k:2]
        idx3 = refs[1:2 * k:2]
        outs = refs[2 * k:3 * k]
        scr = refs[3 * k:]
        idx_all = [scr[7 * j] for j in range(k)]
        rows = [[scr[7 * j + 1 + 3 * b] for b in range(2)] for j in range(k)]
        semg = [[scr[7 * j + 2 + 3 * b] for b in range(2)] for j in range(k)]
        semw = [[scr[7 * j + 3 + 3 * b] for b in range(2)] for j in range(k)]
        wid = lax.axis_index("s") * NC + lax.axis_index("c")
        for j in range(k):
            pltpu.sync_copy(idx3[j].at[:, wid], idx_all[j])

        def trip(t, b):
            tm1, tm2 = t - 1, t - 2
            for j in range(k):
                @pl.when(_valid(wid, tm2, NW))
                def _(j=j):
                    _wait(rows[j][b], outs[j].at[pl.ds(0, CH)], semw[j][b])
            for j in range(k):
                @pl.when(_valid(wid, t, NW))
                def _(j=j):
                    pltpu.async_copy(tabs[j].at[idx_all[j].at[t]],
                                     rows[j][b], semg[j][b])
            for j in range(k):
                @pl.when(_valid(wid, tm1, NW))
                def _(j=j):
                    bp = 1 - b
                    _wait(tabs[j].at[pl.ds(0, CH)], rows[j][bp], semg[j][bp])
                    base = (wid + tm1 * NW) * CH
                    pltpu.async_copy(rows[j][bp], outs[j].at[pl.ds(base, CH)],
                                     semw[j][bp])

        def pair(i, carry):
            trip(2 * i, 0)
            trip(2 * i + 1, 1)
            return carry

        lax.fori_loop(0, (TRIPS32 + 3) // 2, pair, 0)

    return gk


def _make_geom_gather():
    scratch = [pltpu.VMEM((N_NODES * 4,), jnp.float32),
               pltpu.VMEM((TRIPS32, CH), jnp.int32),
               pltpu.VMEM((TRIPS32, CH), jnp.int32),
               pltpu.VMEM((CH, 128), jnp.float32),
               pltpu.VMEM((CH, 128), jnp.float32),
               pltpu.VMEM((CH,), jnp.float32),
               pltpu.SemaphoreType.DMA,
               pltpu.SemaphoreType.DMA,
               pltpu.SemaphoreType.DMA,
               pltpu.SemaphoreType.DMA]
    out_type = [jax.ShapeDtypeStruct((N_EDGES, 128), jnp.float32),
                jax.ShapeDtypeStruct((N_EDGES,), jnp.float32)]

    @functools.partial(pl.kernel, out_type=out_type, mesh=_SC_MESH,
                       scratch_types=scratch,
                       compiler_params=pltpu.CompilerParams(
                           needs_layout_passes=False))
    def gk(hn_tab, pos4, row3, col3, ghn_out, d2_out,
           pos_v, ridx_all, cidx_all, rows0, rows1, d2_v,
           semg0, semg1, semw0, semw1):
        rows = [rows0, rows1]
        semg = [semg0, semg1]
        semw = [semw0, semw1]
        wid = lax.axis_index("s") * NC + lax.axis_index("c")
        pltpu.sync_copy(pos4, pos_v)
        pltpu.sync_copy(row3.at[:, wid], ridx_all)
        pltpu.sync_copy(col3.at[:, wid], cidx_all)

        def trip(t, b):
            tm1, tm2 = t - 1, t - 2

            @pl.when(_valid(wid, tm2, NW))
            def _():
                _wait(rows[b], ghn_out.at[pl.ds(0, CH)], semw[b])

            @pl.when(_valid(wid, t, NW))
            def _():
                pltpu.async_copy(hn_tab.at[cidx_all.at[t]], rows[b], semg[b])
                for l in range(CH // 16):
                    ri = ridx_all[t, pl.ds(l * 16, 16)] * 4
                    ci = cidx_all[t, pl.ds(l * 16, 16)] * 4
                    dx = (plsc.load_gather(pos_v, [ri])
                          - plsc.load_gather(pos_v, [ci]))
                    dy = (plsc.load_gather(pos_v, [ri + 1])
                          - plsc.load_gather(pos_v, [ci + 1]))
                    dz = (plsc.load_gather(pos_v, [ri + 2])
                          - plsc.load_gather(pos_v, [ci + 2]))
                    d2_v[pl.ds(l * 16, 16)] = (dx * dx + dy * dy + dz * dz
                                               + 1e-8)
                pltpu.sync_copy(d2_v,
                                d2_out.at[pl.ds((wid + t * NW) * CH, CH)])

            @pl.when(_valid(wid, tm1, NW))
            def _():
                bp = 1 - b
                _wait(hn_tab.at[pl.ds(0, CH)], rows[bp], semg[bp])
                base = (wid + tm1 * NW) * CH
                pltpu.async_copy(rows[bp], ghn_out.at[pl.ds(base, CH)],
                                 semw[bp])

        def pair(i, carry):
            trip(2 * i, 0)
            trip(2 * i + 1, 1)
            return carry

        lax.fori_loop(0, (TRIPS32 + 3) // 2, pair, 0)

    return gk


def _scatter_loop(vals, idx_all, accum, rows, semv, wid, ways, trips):
    def fire(t, b):
        @pl.when(_valid(wid, t, ways))
        def _():
            base = (wid + t * ways) * CH
            pltpu.async_copy(vals.at[pl.ds(base, CH)], rows[b], semv[b])

    fire(0, 0)
    fire(1, 1)

    def trip(t, b):
        @pl.when(_valid(wid, t, ways))
        def _():
            _wait(vals.at[pl.ds(0, CH)], rows[b], semv[b])
            pltpu.sync_copy(rows[b], accum.at[idx_all.at[t]], add=True)
        fire(t + 2, b)

    def pair(i, carry):
        trip(2 * i, 0)
        trip(2 * i + 1, 1)
        return carry

    lax.fori_loop(0, (trips + 1) // 2, pair, 0)


def _make_scatter1():
    scratch = [pltpu.VMEM((TRIPS32, CH), jnp.int32),
               pltpu.VMEM((CH, 128), jnp.float32),
               pltpu.VMEM((CH, 128), jnp.float32),
               pltpu.SemaphoreType.DMA,
               pltpu.SemaphoreType.DMA,
               pltpu.VMEM_SHARED((N_PAD, 128), jnp.float32)]
    out_type = jax.ShapeDtypeStruct((NC, N_PAD, 128), jnp.float32)

    @functools.partial(pl.kernel, out_type=out_type, mesh=_SC_MESH,
                       scratch_types=scratch)
    def sk(vals, idx3, zeros, out, idx_all, rows0, rows1, semv0, semv1, accum):
        c = lax.axis_index("c")
        s = lax.axis_index("s")
        wid = s * NC + c
        base_n = s * NODE_SLICE
        pltpu.sync_copy(zeros.at[pl.ds(base_n, NODE_SLICE)],
                        accum.at[pl.ds(base_n, NODE_SLICE)])
        pltpu.sync_copy(idx3.at[:, wid], idx_all)
        plsc.subcore_barrier()
        _scatter_loop(vals, idx_all, accum, [rows0, rows1],
                      [semv0, semv1], wid, NW, TRIPS32)
        plsc.subcore_barrier()
        pltpu.sync_copy(accum.at[pl.ds(base_n, NODE_SLICE)],
                        out.at[c, pl.ds(base_n, NODE_SLICE)])

    return sk


def _make_scatter2():
    scratch = [pltpu.VMEM((TRIPS16, CH), jnp.int32),
               pltpu.VMEM((CH, 128), jnp.float32),
               pltpu.VMEM((CH, 128), jnp.float32),
               pltpu.SemaphoreType.DMA,
               pltpu.SemaphoreType.DMA,
               pltpu.VMEM_SHARED((N_PAD, 128), jnp.float32)]
    out_type = jax.ShapeDtypeStruct((NC, N_PAD, 128), jnp.float32)

    @functools.partial(pl.kernel, out_type=out_type, mesh=_SC_MESH,
                       scratch_types=scratch)
    def sk(vl, il3, vr, ir3, zeros, out, idx_all, rows0, rows1,
           semv0, semv1, accum):
        c = lax.axis_index("c")
        s = lax.axis_index("s")
        base_n = s * NODE_SLICE
        pltpu.sync_copy(zeros.at[pl.ds(base_n, NODE_SLICE)],
                        accum.at[pl.ds(base_n, NODE_SLICE)])

        def run(vals, idx3):
            pltpu.sync_copy(idx3.at[:, s], idx_all)
            plsc.subcore_barrier()
            _scatter_loop(vals, idx_all, accum, [rows0, rows1],
                          [semv0, semv1], s, NS, TRIPS16)
            plsc.subcore_barrier()
            pltpu.sync_copy(accum.at[pl.ds(base_n, NODE_SLICE)],
                            out.at[c, pl.ds(base_n, NODE_SLICE)])

        @pl.when(c == 0)
        def _():
            run(vl, il3)

        @pl.when(c == 1)
        def _():
            run(vr, ir3)

    return sk


def _full_spec(shape):
    return pl.BlockSpec(shape, lambda i: tuple(0 for _ in shape))


def _row_spec(block_rows, ncols):
    return pl.BlockSpec((block_rows, ncols), lambda i: (i, 0))


def _ln(x, g, b):
    m = jnp.mean(x, -1, keepdims=True)
    xc = x - m
    v = jnp.mean(xc * xc, -1, keepdims=True)
    return xc * jax.lax.rsqrt(v + 1e-5) * g + b


def _mm(x, w, b=None):
    y = jax.lax.dot_general(x.astype(jnp.bfloat16), w.astype(jnp.bfloat16),
                            (((1,), (0,)), ((), ())),
                            preferred_element_type=jnp.float32)
    if b is not None:
        y = y + b
    return y


def _node_pre_body(x_ref, wl, bl, w1, b1, w2, b2, wc, bc,
                   h_ref, hn_ref, cent_ref):
    x = x_ref[...]
    h = _mm(x, wl[...], bl[...])
    h_ref[...] = h
    t = jnp.maximum(_mm(h, w1[...], b1[...]), 0.0)
    hn_ref[...] = _mm(t, w2[...], b2[...])
    cent_ref[...] = _mm(h, wc[...], bc[...])


def _edge_nbe_body(d2_ref, ghn_ref, off, we, be, w1, b1, w2, b2,
                   wm, bm, msg_ref):
    ghn = ghn_ref[...]
    d = jnp.sqrt(d2_ref[...])
    step = CUTOFF / (NUM_GAUSS - 1)
    coeff = -0.5 / step ** 2
    diff = d - off[...]
    smear = jnp.exp(coeff * diff * diff)
    ea = _mm(smear, we[...], be[...])
    t = jnp.maximum(_mm(ea, w1[...], b1[...]), 0.0)
    he = _mm(t, w2[...], b2[...])
    msg_ref[...] = _mm(he * ghn, wm[...], bm[...])


def _node_post_body(h_ref, cent_ref, agg_ref, lng, lnb, wo, bo,
                    w1, b1, w2, b2, wc2, bc2,
                    h2_ref, hn2_ref, cent2_ref):
    out = cent_ref[...] + agg_ref[0] + agg_ref[1]
    out = _ln(out, lng[...], lnb[...])
    h2 = h_ref[...] + _mm(jnp.maximum(out, 0.0), wo[...], bo[...])
    h2_ref[...] = h2
    t = jnp.maximum(_mm(h2, w1[...], b1[...]), 0.0)
    hn2_ref[...] = _mm(t, w2[...], b2[...])
    cent2_ref[...] = _mm(h2, wc2[...], bc2[...])


def _edge_bond1_body(hb_ref, gl_ref, gr_ref,
                     wbl, wnl, w1l, b1l, w2l, b2l,
                     wbr, wnr, w1r, b1r, w2r, b2r,
                     wfl, bfl, wfr, bfr, ws, bs,
                     ml_ref, mr_ref, part_ref):
    hb = hb_ref[...]
    gl = gl_ref[...]
    gr = gr_ref[...]
    il = _mm(hb, wbl[...]) * _mm(gl, wnl[...])
    t = jnp.maximum(_mm(il, w1l[...], b1l[...]), 0.0)
    ml_ref[...] = _mm(t, w2l[...], b2l[...])
    ir = _mm(hb, wbr[...]) * _mm(gr, wnr[...])
    t = jnp.maximum(_mm(ir, w1r[...], b1r[...]), 0.0)
    mr_ref[...] = _mm(t, w2r[...], b2r[...])
    part_ref[...] = (_mm(gl, wfl[...], bfl[...]) + _mm(gr, wfr[...], bfr[...])
                     + _mm(hb, ws[...], bs[...])).astype(jnp.bfloat16)


def _edge_bond2_body(hb_ref, ga1_ref, gac_ref, part_ref,
                     lng, lnb, wo, bo, w1, b1, w2, b2, wm, bm,
                     hb2_ref, msg2_ref):
    gac = gac_ref[...]
    ga2 = jax.lax.bitcast_convert_type(gac << 16, jnp.float32)
    ghn2 = jax.lax.bitcast_convert_type(
        gac & jnp.int32(-65536), jnp.float32)
    pre = ga1_ref[...] + ga2 + part_ref[...].astype(jnp.float32)
    pre = _ln(pre, lng[...], lnb[...])
    hb2 = (hb_ref[...].astype(jnp.float32)
           + _mm(jnp.maximum(pre, 0.0), wo[...], bo[...]))
    hb2_ref[...] = hb2
    t = jnp.maximum(_mm(hb2, w1[...], b1[...]), 0.0)
    he2 = _mm(t, w2[...], b2[...])
    msg2_ref[...] = _mm(he2 * ghn2, wm[...], bm[...])


def _node_final_body(h2_ref, cent2_ref, agg_ref, lng, lnb, wo, bo,
                     h3_ref):
    out = cent2_ref[...] + agg_ref[0] + agg_ref[1]
    out = _ln(out, lng[...], lnb[...])
    h3_ref[...] = h2_ref[...] + _mm(jnp.maximum(out, 0.0), wo[...], bo[...])


def _tc_call(body, grid, in_arrs, in_specs, out_shapes, out_specs):
    return pl.pallas_call(
        body,
        grid=(grid,),
        in_specs=in_specs,
        out_specs=out_specs,
        out_shape=out_shapes,
    )(*in_arrs)


def _agg_spec():
    return pl.BlockSpec((2, TN, 128), lambda i: (0, i, 0))


_gather_g12 = _make_geom_gather()
_gather_g3 = _make_gather(((128, jnp.float32), (128, jnp.float32)))
_gather_g4 = _make_gather(((128, jnp.float32), (128, jnp.int32)))
_scatter_s1 = _make_scatter1()
_scatter_s2 = _make_scatter2()


def _b2(v):
    return v.reshape(1, -1)


def _idx3(idx, ways, trips):
    pad = trips * ways * CH - N_EDGES
    return jnp.pad(idx, (0, pad)).reshape(trips, ways, CH)


def kernel(h_node, pos_node, h_bond, bond_index, batch, is_mol, is_frag, params):
    P = params
    nbe = P["nbe"][0]
    nbb = P["nbb"][0]
    bb = P["bb"][0]
    row = bond_index[0]
    col = bond_index[1]
    row32 = _idx3(row, NW, TRIPS32)
    col32 = _idx3(col, NW, TRIPS32)
    row16 = _idx3(row, NS, TRIPS16)
    col16 = _idx3(col, NS, TRIPS16)
    zeros_n = jnp.zeros((N_PAD, 128), jnp.float32)

    pos4 = jnp.zeros((N_NODES, 4), jnp.float32).at[:, :3].set(pos_node)
    pos4 = pos4.reshape(N_NODES * 4)
    off = np.zeros((1, 32), np.float32)
    off[0, :NUM_GAUSS] = np.linspace(0.0, CUTOFF, NUM_GAUSS)
    off = jnp.asarray(off)
    we_pad = jnp.zeros((32, 128), jnp.float32).at[:NUM_GAUSS].set(P["edge_emb"]["W"])

    ew = _full_spec

    h, hn_all, cent = _tc_call(
        _node_pre_body, N_NODES // TN,
        [h_node, P["lin_node"]["W"], _b2(P["lin_node"]["b"]),
         nbe["node_net"]["l1"]["W"], _b2(nbe["node_net"]["l1"]["b"]),
         nbe["node_net"]["l2"]["W"], _b2(nbe["node_net"]["l2"]["b"]),
         nbe["centroid"]["W"], _b2(nbe["centroid"]["b"])],
        [_row_spec(TN, 128)] + [ew((128, 128)), ew((1, 128))] * 4,
        [jax.ShapeDtypeStruct((N_NODES, 128), jnp.float32)] * 3,
        [_row_spec(TN, 128)] * 3,
    )

    ghn, d2 = _gather_g12(hn_all, pos4, row32, col32)
    d2 = d2.reshape(N_EDGES, 1)

    (msg,) = _tc_call(
        _edge_nbe_body, N_EDGES // TE,
        [d2, ghn, off, we_pad, _b2(P["edge_emb"]["b"]),
         nbe["edge_net"]["l1"]["W"], _b2(nbe["edge_net"]["l1"]["b"]),
         nbe["edge_net"]["l2"]["W"], _b2(nbe["edge_net"]["l2"]["b"]),
         nbe["msg_net"]["W"], _b2(nbe["msg_net"]["b"])],
        [_row_spec(TE, 1), _row_spec(TE, 128),
         ew((1, 32)), ew((32, 128)), ew((1, 128)),
         ew((128, 128)), ew((1, 128)), ew((128, 128)), ew((1, 128)),
         ew((128, 128)), ew((1, 128))],
        [jax.ShapeDtypeStruct((N_EDGES, 128), jnp.float32)],
        [_row_spec(TE, 128)],
    )

    aggr = _scatter_s1(msg, row32, zeros_n)

    h2, hn2, cent2 = _tc_call(
        _node_post_body, N_NODES // TN,
        [h, cent, aggr, _b2(nbe["ln_g"]), _b2(nbe["ln_b"]),
         nbe["out"]["W"], _b2(nbe["out"]["b"]),
         nbb["node_net"]["l1"]["W"], _b2(nbb["node_net"]["l1"]["b"]),
         nbb["node_net"]["l2"]["W"], _b2(nbb["node_net"]["l2"]["b"]),
         nbb["centroid"]["W"], _b2(nbb["centroid"]["b"])],
        [_row_spec(TN, 128)] * 2 + [_agg_spec()]
        + [ew((1, 128)), ew((1, 128))]
        + [ew((128, 128)), ew((1, 128))] * 4,
        [jax.ShapeDtypeStruct((N_NODES, 128), jnp.float32)] * 3,
        [_row_spec(TN, 128)] * 3,
    )

    gl, gr = _gather_g3(h2, row32, h2, col32)

    hb16 = h_bond.astype(jnp.bfloat16)
    m_l_pre, m_r_pre, part = _tc_call(
        _edge_bond1_body, N_EDGES // TE,
        [hb16, gl, gr,
         bb["ffn_l"]["bond_lin"]["W"], bb["ffn_l"]["node_lin"]["W"],
         bb["ffn_l"]["inter"]["l1"]["W"], _b2(bb["ffn_l"]["inter"]["l1"]["b"]),
         bb["ffn_l"]["inter"]["l2"]["W"], _b2(bb["ffn_l"]["inter"]["l2"]["b"]),
         bb["ffn_r"]["bond_lin"]["W"], bb["ffn_r"]["node_lin"]["W"],
         bb["ffn_r"]["inter"]["l1"]["W"], _b2(bb["ffn_r"]["inter"]["l1"]["b"]),
         bb["ffn_r"]["inter"]["l2"]["W"], _b2(bb["ffn_r"]["inter"]["l2"]["b"]),
         bb["node_ffn_l"]["W"], _b2(bb["node_ffn_l"]["b"]),
         bb["node_ffn_r"]["W"], _b2(bb["node_ffn_r"]["b"]),
         bb["self_ffn"]["W"], _b2(bb["self_ffn"]["b"])],
        [_row_spec(TE, 128)] * 3
        + [ew((128, 256)), ew((128, 256)), ew((256, 256)), ew((1, 256)),
           ew((256, 128)), ew((1, 128))] * 2
        + [ew((128, 128)), ew((1, 128))] * 3,
        [jax.ShapeDtypeStruct((N_EDGES, 128), jnp.float32)] * 2
        + [jax.ShapeDtypeStruct((N_EDGES, 128), jnp.bfloat16)],
        [_row_spec(TE, 128)] * 3,
    )

    A12 = _scatter_s2(m_l_pre, col16, m_r_pre, row16, zeros_n)
    A1 = A12[0]
    a2b = jax.lax.bitcast_convert_type(A12[1][:N_NODES], jnp.uint32)
    hnb = jax.lax.bitcast_convert_type(hn2, jnp.uint32)
    tac = jax.lax.bitcast_convert_type(
        (hnb & jnp.uint32(0xFFFF0000)) | (a2b >> 16), jnp.int32)

    gA1, gac = _gather_g4(A1, row32, tac, col32)

    hb2, msg2 = _tc_call(
        _edge_bond2_body, N_EDGES // TE,
        [hb16, gA1, gac, part,
         _b2(bb["ln_g"]), _b2(bb["ln_b"]),
         bb["out"]["W"], _b2(bb["out"]["b"]),
         nbb["edge_net"]["l1"]["W"], _b2(nbb["edge_net"]["l1"]["b"]),
         nbb["edge_net"]["l2"]["W"], _b2(nbb["edge_net"]["l2"]["b"]),
         nbb["msg_net"]["W"], _b2(nbb["msg_net"]["b"])],
        [_row_spec(TE, 128)] * 4
        + [ew((1, 128)), ew((1, 128))]
        + [ew((128, 128)), ew((1, 128))] * 4,
        [jax.ShapeDtypeStruct((N_EDGES, 128), jnp.float32)] * 2,
        [_row_spec(TE, 128)] * 2,
    )

    aggr2 = _scatter_s1(msg2, row32, zeros_n)

    (h3,) = _tc_call(
        _node_final_body, N_NODES // TN,
        [h2, cent2, aggr2, _b2(nbb["ln_g"]), _b2(nbb["ln_b"]),
         nbb["out"]["W"], _b2(nbb["out"]["b"])],
        [_row_spec(TN, 128)] * 2 + [_agg_spec()]
        + [ew((1, 128)), ew((1, 128)), ew((128, 128)), ew((1, 128))],
        [jax.ShapeDtypeStruct((N_NODES, 128), jnp.float32)],
        [_row_spec(TN, 128)],
    )

    return h3, hb2


# speedup vs baseline: 4.6403x; 1.0194x over previous
import functools
import jax
import jax.numpy as jnp
import numpy as np
from jax import lax
from jax.experimental import pallas as pl
from jax.experimental.pallas import tpu as pltpu
from jax.experimental.pallas import tpu_sc as plsc

N_NODES = 10000
N_EDGES = 160000
NUM_GAUSS = 20
CUTOFF = 10.0

TE = 6400
TN = 1000

NC, NS = 2, 16
NW = NC * NS
CH = 128
NCHUNK = N_EDGES // CH
TRIPS32 = -(-NCHUNK // NW)
TRIPS16 = -(-NCHUNK // NS)
N_PAD = 10240
NODE_SLICE = N_PAD // NS

_SC_MESH = plsc.VectorSubcoreMesh(core_axis_name="c", subcore_axis_name="s")


def _valid(wid, t, ways):
    return jnp.logical_and(t >= 0, wid + t * ways < NCHUNK)


def _validn(wid, t, nch):
    return jnp.logical_and(t >= 0, wid + t * NW < nch)


def _wait(src, dst, sem):
    pltpu.make_async_copy(src, dst, sem).wait()


def _make_gather(dims, nch=NCHUNK):
    k = len(dims)
    trips = -(-nch // NW)
    scratch = []
    for d, dt in dims:
        scratch.append(pltpu.VMEM((trips, CH), jnp.int32))
        for b in range(2):
            scratch.append(pltpu.VMEM((CH, d), dt))
            scratch.append(pltpu.SemaphoreType.DMA)
            scratch.append(pltpu.SemaphoreType.DMA)
    out_type = [jax.ShapeDtypeStruct((nch * CH, d), dt) for (d, dt) in dims]

    @functools.partial(pl.kernel, out_type=out_type, mesh=_SC_MESH,
                       scratch_types=scratch)
    def gk(*refs):
        tabs = refs[0:2 * k:2]
        idx3 = refs[1:2 * k:2]
        outs = refs[2 * k:3 * k]
        scr = refs[3 * k:]
        idx_all = [scr[7 * j] for j in range(k)]
        rows = [[scr[7 * j + 1 + 3 * b] for b in range(2)] for j in range(k)]
        semg = [[scr[7 * j + 2 + 3 * b] for b in range(2)] for j in range(k)]
        semw = [[scr[7 * j + 3 + 3 * b] for b in range(2)] for j in range(k)]
        wid = lax.axis_index("s") * NC + lax.axis_index("c")
        for j in range(k):
            pltpu.sync_copy(idx3[j].at[:, wid], idx_all[j])

        def trip(t, b):
            tm1, tm2 = t - 1, t - 2
            for j in range(k):
                @pl.when(_validn(wid, tm2, nch))
                def _(j=j):
                    _wait(rows[j][b], outs[j].at[pl.ds(0, CH)], semw[j][b])
            for j in range(k):
                @pl.when(_validn(wid, t, nch))
                def _(j=j):
                    pltpu.async_copy(tabs[j].at[idx_all[j].at[t]],
                                     rows[j][b], semg[j][b])
            for j in range(k):
                @pl.when(_validn(wid, tm1, nch))
                def _(j=j):
                    bp = 1 - b
                    _wait(tabs[j].at[pl.ds(0, CH)], rows[j][bp], semg[j][bp])
                    base = (wid + tm1 * NW) * CH
                    pltpu.async_copy(rows[j][bp], outs[j].at[pl.ds(base, CH)],
                                     semw[j][bp])

        def pair(i, carry):
            trip(2 * i, 0)
            trip(2 * i + 1, 1)
            return carry

        lax.fori_loop(0, (trips + 3) // 2, pair, 0)

    return gk


def _make_geom_gather(nch=NCHUNK):
    trips = -(-nch // NW)
    scratch = [pltpu.VMEM((N_NODES * 4,), jnp.float32),
               pltpu.VMEM((trips, CH), jnp.int32),
               pltpu.VMEM((trips, CH), jnp.int32),
               pltpu.VMEM((CH, 128), jnp.float32),
               pltpu.VMEM((CH, 128), jnp.float32),
               pltpu.VMEM((CH,), jnp.float32),
               pltpu.SemaphoreType.DMA,
               pltpu.SemaphoreType.DMA,
               pltpu.SemaphoreType.DMA,
               pltpu.SemaphoreType.DMA]
    out_type = [jax.ShapeDtypeStruct((nch * CH, 128), jnp.float32),
                jax.ShapeDtypeStruct((nch * CH,), jnp.float32)]

    @functools.partial(pl.kernel, out_type=out_type, mesh=_SC_MESH,
                       scratch_types=scratch,
                       compiler_params=pltpu.CompilerParams(
                           needs_layout_passes=False))
    def gk(hn_tab, pos4, row3, col3, ghn_out, d2_out,
           pos_v, ridx_all, cidx_all, rows0, rows1, d2_v,
           semg0, semg1, semw0, semw1):
        rows = [rows0, rows1]
        semg = [semg0, semg1]
        semw = [semw0, semw1]
        wid = lax.axis_index("s") * NC + lax.axis_index("c")
        pltpu.sync_copy(pos4, pos_v)
        pltpu.sync_copy(row3.at[:, wid], ridx_all)
        pltpu.sync_copy(col3.at[:, wid], cidx_all)

        def trip(t, b):
            tm1, tm2 = t - 1, t - 2

            @pl.when(_validn(wid, tm2, nch))
            def _():
                _wait(rows[b], ghn_out.at[pl.ds(0, CH)], semw[b])

            @pl.when(_validn(wid, t, nch))
            def _():
                pltpu.async_copy(hn_tab.at[cidx_all.at[t]], rows[b], semg[b])
                for l in range(CH // 16):
                    ri = ridx_all[t, pl.ds(l * 16, 16)] * 4
                    ci = cidx_all[t, pl.ds(l * 16, 16)] * 4
                    dx = (plsc.load_gather(pos_v, [ri])
                          - plsc.load_gather(pos_v, [ci]))
                    dy = (plsc.load_gather(pos_v, [ri + 1])
                          - plsc.load_gather(pos_v, [ci + 1]))
                    dz = (plsc.load_gather(pos_v, [ri + 2])
                          - plsc.load_gather(pos_v, [ci + 2]))
                    d2_v[pl.ds(l * 16, 16)] = (dx * dx + dy * dy + dz * dz
                                               + 1e-8)
                pltpu.sync_copy(d2_v,
                                d2_out.at[pl.ds((wid + t * NW) * CH, CH)])

            @pl.when(_validn(wid, tm1, nch))
            def _():
                bp = 1 - b
                _wait(hn_tab.at[pl.ds(0, CH)], rows[bp], semg[bp])
                base = (wid + tm1 * NW) * CH
                pltpu.async_copy(rows[bp], ghn_out.at[pl.ds(base, CH)],
                                 semw[bp])

        def pair(i, carry):
            trip(2 * i, 0)
            trip(2 * i + 1, 1)
            return carry

        lax.fori_loop(0, (trips + 3) // 2, pair, 0)

    return gk


def _scatter_loop(halves, idx_all, accum, rows, semv, wid, ways, trips):
    def fire(t, b):
        chunk = wid + t * ways
        for v, lo, n in halves:
            @pl.when(jnp.logical_and(
                _valid(wid, t, ways),
                jnp.logical_and(chunk >= lo, chunk < lo + n)))
            def _(v=v, lo=lo):
                pltpu.async_copy(v.at[pl.ds((chunk - lo) * CH, CH)],
                                 rows[b], semv[b])

    fire(0, 0)
    fire(1, 1)

    def trip(t, b):
        @pl.when(_valid(wid, t, ways))
        def _():
            _wait(halves[0][0].at[pl.ds(0, CH)], rows[b], semv[b])
            pltpu.sync_copy(rows[b], accum.at[idx_all.at[t]], add=True)
        fire(t + 2, b)

    def pair(i, carry):
        trip(2 * i, 0)
        trip(2 * i + 1, 1)
        return carry

    lax.fori_loop(0, (trips + 1) // 2, pair, 0)


def _make_scatter1(nsrc):
    scratch = [pltpu.VMEM((TRIPS32, CH), jnp.int32),
               pltpu.VMEM((CH, 128), jnp.float32),
               pltpu.VMEM((CH, 128), jnp.float32),
               pltpu.SemaphoreType.DMA,
               pltpu.SemaphoreType.DMA,
               pltpu.VMEM_SHARED((N_PAD, 128), jnp.float32)]
    out_type = jax.ShapeDtypeStruct((NC, N_PAD, 128), jnp.float32)

    @functools.partial(pl.kernel, out_type=out_type, mesh=_SC_MESH,
                       scratch_types=scratch)
    def sk(*refs):
        vals = refs[:nsrc]
        idx3, zeros, out = refs[nsrc:nsrc + 3]
        idx_all, rows0, rows1, semv0, semv1, accum = refs[nsrc + 3:]
        if nsrc == 1:
            halves = [(vals[0], 0, NCHUNK)]
        else:
            halves = [(vals[0], 0, NCHUNK // 2),
                      (vals[1], NCHUNK // 2, NCHUNK // 2)]
        c = lax.axis_index("c")
        s = lax.axis_index("s")
        wid = s * NC + c
        base_n = s * NODE_SLICE
        pltpu.sync_copy(zeros.at[pl.ds(base_n, NODE_SLICE)],
                        accum.at[pl.ds(base_n, NODE_SLICE)])
        pltpu.sync_copy(idx3.at[:, wid], idx_all)
        plsc.subcore_barrier()
        _scatter_loop(halves, idx_all, accum, [rows0, rows1],
                      [semv0, semv1], wid, NW, TRIPS32)
        plsc.subcore_barrier()
        pltpu.sync_copy(accum.at[pl.ds(base_n, NODE_SLICE)],
                        out.at[c, pl.ds(base_n, NODE_SLICE)])

    return sk


def _make_scatter2():
    scratch = [pltpu.VMEM((TRIPS16, CH), jnp.int32),
               pltpu.VMEM((CH, 128), jnp.float32),
               pltpu.VMEM((CH, 128), jnp.float32),
               pltpu.SemaphoreType.DMA,
               pltpu.SemaphoreType.DMA,
               pltpu.VMEM_SHARED((N_PAD, 128), jnp.float32)]
    out_type = jax.ShapeDtypeStruct((NC, N_PAD, 128), jnp.float32)

    @functools.partial(pl.kernel, out_type=out_type, mesh=_SC_MESH,
                       scratch_types=scratch)
    def sk(vla, vlb, il3, vra, vrb, ir3, zeros, out, idx_all, rows0, rows1,
           semv0, semv1, accum):
        c = lax.axis_index("c")
        s = lax.axis_index("s")
        base_n = s * NODE_SLICE
        pltpu.sync_copy(zeros.at[pl.ds(base_n, NODE_SLICE)],
                        accum.at[pl.ds(base_n, NODE_SLICE)])

        def run(va, vb, idx3):
            pltpu.sync_copy(idx3.at[:, s], idx_all)
            plsc.subcore_barrier()
            _scatter_loop([(va, 0, NCHUNK // 2),
                           (vb, NCHUNK // 2, NCHUNK // 2)],
                          idx_all, accum, [rows0, rows1],
                          [semv0, semv1], s, NS, TRIPS16)
            plsc.subcore_barrier()
            pltpu.sync_copy(accum.at[pl.ds(base_n, NODE_SLICE)],
                            out.at[c, pl.ds(base_n, NODE_SLICE)])

        @pl.when(c == 0)
        def _():
            run(vla, vlb, il3)

        @pl.when(c == 1)
        def _():
            run(vra, vrb, ir3)

    return sk


def _full_spec(shape):
    return pl.BlockSpec(shape, lambda i: tuple(0 for _ in shape))


def _row_spec(block_rows, ncols):
    return pl.BlockSpec((block_rows, ncols), lambda i: (i, 0))


def _ln(x, g, b):
    m = jnp.mean(x, -1, keepdims=True)
    xc = x - m
    v = jnp.mean(xc * xc, -1, keepdims=True)
    return xc * jax.lax.rsqrt(v + 1e-5) * g + b


def _mm(x, w, b=None):
    y = jax.lax.dot_general(x.astype(jnp.bfloat16), w.astype(jnp.bfloat16),
                            (((1,), (0,)), ((), ())),
                            preferred_element_type=jnp.float32)
    if b is not None:
        y = y + b
    return y


def _node_pre_body(x_ref, wl, bl, w1, b1, w2, b2, wc, bc,
                   h_ref, hn_ref, cent_ref):
    x = x_ref[...]
    h = _mm(x, wl[...], bl[...])
    h_ref[...] = h
    t = jnp.maximum(_mm(h, w1[...], b1[...]), 0.0)
    hn_ref[...] = _mm(t, w2[...], b2[...])
    cent_ref[...] = _mm(h, wc[...], bc[...])


def _edge_nbe_body(d2_ref, ghn_ref, off, we, be, w1, b1, w2, b2,
                   wm, bm, msg_ref):
    ghn = ghn_ref[...]
    d = jnp.sqrt(d2_ref[...])
    step = CUTOFF / (NUM_GAUSS - 1)
    coeff = -0.5 / step ** 2
    diff = d - off[...]
    smear = jnp.exp(coeff * diff * diff)
    ea = _mm(smear, we[...], be[...])
    t = jnp.maximum(_mm(ea, w1[...], b1[...]), 0.0)
    he = _mm(t, w2[...], b2[...])
    msg_ref[...] = _mm(he * ghn, wm[...], bm[...])


def _node_post_body(h_ref, cent_ref, agg_ref, lng, lnb, wo, bo,
                    w1, b1, w2, b2, wc2, bc2,
                    h2_ref, hn2_ref, cent2_ref):
    out = cent_ref[...] + agg_ref[0] + agg_ref[1]
    out = _ln(out, lng[...], lnb[...])
    h2 = h_ref[...] + _mm(jnp.maximum(out, 0.0), wo[...], bo[...])
    h2_ref[...] = h2
    t = jnp.maximum(_mm(h2, w1[...], b1[...]), 0.0)
    hn2_ref[...] = _mm(t, w2[...], b2[...])
    cent2_ref[...] = _mm(h2, wc2[...], bc2[...])


def _edge_bond1_body(hb_ref, gl_ref, gr_ref,
                     wbl, wnl, w1l, b1l, w2l, b2l,
                     wbr, wnr, w1r, b1r, w2r, b2r,
                     wfl, bfl, wfr, bfr, ws, bs,
                     ml_ref, mr_ref, part_ref):
    hb = hb_ref[...]
    gl = gl_ref[...]
    gr = gr_ref[...]
    il = _mm(hb, wbl[...]) * _mm(gl, wnl[...])
    t = jnp.maximum(_mm(il, w1l[...], b1l[...]), 0.0)
    ml_ref[...] = _mm(t, w2l[...], b2l[...])
    ir = _mm(hb, wbr[...]) * _mm(gr, wnr[...])
    t = jnp.maximum(_mm(ir, w1r[...], b1r[...]), 0.0)
    mr_ref[...] = _mm(t, w2r[...], b2r[...])
    part_ref[...] = (_mm(gl, wfl[...], bfl[...]) + _mm(gr, wfr[...], bfr[...])
                     + _mm(hb, ws[...], bs[...])).astype(jnp.bfloat16)


def _edge_bond2_body(hb_ref, ga1_ref, gac_ref, parta_ref, partb_ref,
                     lng, lnb, wo, bo, w1, b1, w2, b2, wm, bm,
                     hb2_ref, msg2_ref):
    gac = gac_ref[...]
    ga2 = jax.lax.bitcast_convert_type(gac << 16, jnp.float32)
    ghn2 = jax.lax.bitcast_convert_type(
        gac & jnp.int32(-65536), jnp.float32)
    nblk_h = EH // TE_H
    part = jnp.where(pl.program_id(0) < nblk_h,
                     parta_ref[...], partb_ref[...])
    pre = ga1_ref[...] + ga2 + part.astype(jnp.float32)
    pre = _ln(pre, lng[...], lnb[...])
    hb2 = (hb_ref[...].astype(jnp.float32)
           + _mm(jnp.maximum(pre, 0.0), wo[...], bo[...]))
    hb2_ref[...] = hb2
    t = jnp.maximum(_mm(hb2, w1[...], b1[...]), 0.0)
    he2 = _mm(t, w2[...], b2[...])
    msg2_ref[...] = _mm(he2 * ghn2, wm[...], bm[...])


def _node_final_body(h2_ref, cent2_ref, agg_ref, lng, lnb, wo, bo,
                     h3_ref):
    out = cent2_ref[...] + agg_ref[0] + agg_ref[1]
    out = _ln(out, lng[...], lnb[...])
    h3_ref[...] = h2_ref[...] + _mm(jnp.maximum(out, 0.0), wo[...], bo[...])


def _tc_call(body, grid, in_arrs, in_specs, out_shapes, out_specs):
    return pl.pallas_call(
        body,
        grid=(grid,),
        in_specs=in_specs,
        out_specs=out_specs,
        out_shape=out_shapes,
    )(*in_arrs)


def _agg_spec():
    return pl.BlockSpec((2, TN, 128), lambda i: (0, i, 0))


NCH_H = NCHUNK // 2
EH = NCH_H * CH
TE_H = 4000

_gather_g12h = _make_geom_gather(NCH_H)
_gather_g3h = _make_gather(((128, jnp.float32), (128, jnp.float32)), NCH_H)
_gather_g4 = _make_gather(((128, jnp.float32), (128, jnp.int32)))
_scatter_s1 = _make_scatter1(2)
_scatter_s3 = _make_scatter1(1)
_scatter_s2 = _make_scatter2()


def _b2(v):
    return v.reshape(1, -1)


def _idx3(idx, ways, trips):
    pad = trips * ways * CH - idx.shape[0]
    return jnp.pad(idx, (0, pad)).reshape(trips, ways, CH)


def _idx3h(idx, lo):
    trips = -(-NCH_H // NW)
    return _idx3(idx[lo * CH:(lo + NCH_H) * CH], NW, trips)


def kernel(h_node, pos_node, h_bond, bond_index, batch, is_mol, is_frag, params):
    P = params
    nbe = P["nbe"][0]
    nbb = P["nbb"][0]
    bb = P["bb"][0]
    row = bond_index[0]
    col = bond_index[1]
    row32 = _idx3(row, NW, TRIPS32)
    col32 = _idx3(col, NW, TRIPS32)
    row32a, row32b = _idx3h(row, 0), _idx3h(row, NCH_H)
    col32a, col32b = _idx3h(col, 0), _idx3h(col, NCH_H)
    row16 = _idx3(row, NS, TRIPS16)
    col16 = _idx3(col, NS, TRIPS16)
    zeros_n = jnp.zeros((N_PAD, 128), jnp.float32)

    pos4 = jnp.zeros((N_NODES, 4), jnp.float32).at[:, :3].set(pos_node)
    pos4 = pos4.reshape(N_NODES * 4)
    off = np.zeros((1, 32), np.float32)
    off[0, :NUM_GAUSS] = np.linspace(0.0, CUTOFF, NUM_GAUSS)
    off = jnp.asarray(off)
    we_pad = jnp.zeros((32, 128), jnp.float32).at[:NUM_GAUSS].set(P["edge_emb"]["W"])

    ew = _full_spec

    h, hn_all, cent = _tc_call(
        _node_pre_body, N_NODES // TN,
        [h_node, P["lin_node"]["W"], _b2(P["lin_node"]["b"]),
         nbe["node_net"]["l1"]["W"], _b2(nbe["node_net"]["l1"]["b"]),
         nbe["node_net"]["l2"]["W"], _b2(nbe["node_net"]["l2"]["b"]),
         nbe["centroid"]["W"], _b2(nbe["centroid"]["b"])],
        [_row_spec(TN, 128)] + [ew((128, 128)), ew((1, 128))] * 4,
        [jax.ShapeDtypeStruct((N_NODES, 128), jnp.float32)] * 3,
        [_row_spec(TN, 128)] * 3,
    )

    msg_h = []
    for r3, c3 in ((row32a, col32a), (row32b, col32b)):
        ghn, d2 = _gather_g12h(hn_all, pos4, r3, c3)
        d2 = d2.reshape(EH, 1)
        (m,) = _tc_call(
            _edge_nbe_body, EH // TE_H,
            [d2, ghn, off, we_pad, _b2(P["edge_emb"]["b"]),
             nbe["edge_net"]["l1"]["W"], _b2(nbe["edge_net"]["l1"]["b"]),
             nbe["edge_net"]["l2"]["W"], _b2(nbe["edge_net"]["l2"]["b"]),
             nbe["msg_net"]["W"], _b2(nbe["msg_net"]["b"])],
            [_row_spec(TE_H, 1), _row_spec(TE_H, 128),
             ew((1, 32)), ew((32, 128)), ew((1, 128)),
             ew((128, 128)), ew((1, 128)), ew((128, 128)), ew((1, 128)),
             ew((128, 128)), ew((1, 128))],
            [jax.ShapeDtypeStruct((EH, 128), jnp.float32)],
            [_row_spec(TE_H, 128)],
        )
        msg_h.append(m)

    aggr = _scatter_s1(msg_h[0], msg_h[1], row32, zeros_n)

    h2, hn2, cent2 = _tc_call(
        _node_post_body, N_NODES // TN,
        [h, cent, aggr, _b2(nbe["ln_g"]), _b2(nbe["ln_b"]),
         nbe["out"]["W"], _b2(nbe["out"]["b"]),
         nbb["node_net"]["l1"]["W"], _b2(nbb["node_net"]["l1"]["b"]),
         nbb["node_net"]["l2"]["W"], _b2(nbb["node_net"]["l2"]["b"]),
         nbb["centroid"]["W"], _b2(nbb["centroid"]["b"])],
        [_row_spec(TN, 128)] * 2 + [_agg_spec()]
        + [ew((1, 128)), ew((1, 128))]
        + [ew((128, 128)), ew((1, 128))] * 4,
        [jax.ShapeDtypeStruct((N_NODES, 128), jnp.float32)] * 3,
        [_row_spec(TN, 128)] * 3,
    )

    hb16 = h_bond.astype(jnp.bfloat16)
    ml_h, mr_h, part_h = [], [], []
    for hidx, (r3, c3) in enumerate(((row32a, col32a), (row32b, col32b))):
        gl, gr = _gather_g3h(h2, r3, h2, c3)
        hb_spec = pl.BlockSpec((TE_H, 128),
                               lambda i, H=hidx: (i + H * (EH // TE_H), 0))
        ml, mr, pt = _tc_call(
            _edge_bond1_body, EH // TE_H,
            [hb16, gl, gr,
             bb["ffn_l"]["bond_lin"]["W"], bb["ffn_l"]["node_lin"]["W"],
             bb["ffn_l"]["inter"]["l1"]["W"], _b2(bb["ffn_l"]["inter"]["l1"]["b"]),
             bb["ffn_l"]["inter"]["l2"]["W"], _b2(bb["ffn_l"]["inter"]["l2"]["b"]),
             bb["ffn_r"]["bond_lin"]["W"], bb["ffn_r"]["node_lin"]["W"],
             bb["ffn_r"]["inter"]["l1"]["W"], _b2(bb["ffn_r"]["inter"]["l1"]["b"]),
             bb["ffn_r"]["inter"]["l2"]["W"], _b2(bb["ffn_r"]["inter"]["l2"]["b"]),
             bb["node_ffn_l"]["W"], _b2(bb["node_ffn_l"]["b"]),
             bb["node_ffn_r"]["W"], _b2(bb["node_ffn_r"]["b"]),
             bb["self_ffn"]["W"], _b2(bb["self_ffn"]["b"])],
            [hb_spec, _row_spec(TE_H, 128), _row_spec(TE_H, 128)]
            + [ew((128, 256)), ew((128, 256)), ew((256, 256)), ew((1, 256)),
               ew((256, 128)), ew((1, 128))] * 2
            + [ew((128, 128)), ew((1, 128))] * 3,
            [jax.ShapeDtypeStruct((EH, 128), jnp.float32)] * 2
            + [jax.ShapeDtypeStruct((EH, 128), jnp.bfloat16)],
            [_row_spec(TE_H, 128)] * 3,
        )
        ml_h.append(ml)
        mr_h.append(mr)
        part_h.append(pt)

    A12 = _scatter_s2(ml_h[0], ml_h[1], col16, mr_h[0], mr_h[1], row16,
                      zeros_n)
    A1 = A12[0]
    a2b = jax.lax.bitcast_convert_type(A12[1][:N_NODES], jnp.uint32)
    hnb = jax.lax.bitcast_convert_type(hn2, jnp.uint32)
    tac = jax.lax.bitcast_convert_type(
        (hnb & jnp.uint32(0xFFFF0000)) | (a2b >> 16), jnp.int32)

    gA1, gac = _gather_g4(A1, row32, tac, col32)

    nblk_h = EH // TE_H
    parta_spec = pl.BlockSpec((TE_H, 128),
                              lambda i: (jnp.minimum(i, nblk_h - 1), 0))
    partb_spec = pl.BlockSpec((TE_H, 128),
                              lambda i: (jnp.maximum(i - nblk_h, 0), 0))
    hb2, msg2 = _tc_call(
        _edge_bond2_body, N_EDGES // TE_H,
        [hb16, gA1, gac, part_h[0], part_h[1],
         _b2(bb["ln_g"]), _b2(bb["ln_b"]),
         bb["out"]["W"], _b2(bb["out"]["b"]),
         nbb["edge_net"]["l1"]["W"], _b2(nbb["edge_net"]["l1"]["b"]),
         nbb["edge_net"]["l2"]["W"], _b2(nbb["edge_net"]["l2"]["b"]),
         nbb["msg_net"]["W"], _b2(nbb["msg_net"]["b"])],
        [_row_spec(TE_H, 128)] * 3 + [parta_spec, partb_spec]
        + [ew((1, 128)), ew((1, 128))]
        + [ew((128, 128)), ew((1, 128))] * 4,
        [jax.ShapeDtypeStruct((N_EDGES, 128), jnp.float32)] * 2,
        [_row_spec(TE_H, 128)] * 2,
    )

    aggr2 = _scatter_s3(msg2, row32, zeros_n)

    (h3,) = _tc_call(
        _node_final_body, N_NODES // TN,
        [h2, cent2, aggr2, _b2(nbb["ln_g"]), _b2(nbb["ln_b"]),
         nbb["out"]["W"], _b2(nbb["out"]["b"])],
        [_row_spec(TN, 128)] * 2 + [_agg_spec()]
        + [ew((1, 128)), ew((1, 128)), ew((128, 128)), ew((1, 128))],
        [jax.ShapeDtypeStruct((N_NODES, 128), jnp.float32)],
        [_row_spec(TN, 128)],
    )

    return h3, hb2


# speedup vs baseline: 4.6545x; 1.0031x over previous
import functools
import jax
import jax.numpy as jnp
import numpy as np
from jax import lax
from jax.experimental import pallas as pl
from jax.experimental.pallas import tpu as pltpu
from jax.experimental.pallas import tpu_sc as plsc

N_NODES = 10000
N_EDGES = 160000
NUM_GAUSS = 20
CUTOFF = 10.0

TE = 6400
TN = 1000

NC, NS = 2, 16
NW = NC * NS
CH = 128
NCHUNK = N_EDGES // CH
TRIPS32 = -(-NCHUNK // NW)
TRIPS16 = -(-NCHUNK // NS)
N_PAD = 10240
NODE_SLICE = N_PAD // NS

_SC_MESH = plsc.VectorSubcoreMesh(core_axis_name="c", subcore_axis_name="s")


def _valid(wid, t, ways):
    return jnp.logical_and(t >= 0, wid + t * ways < NCHUNK)


def _validn(wid, t, nch):
    return jnp.logical_and(t >= 0, wid + t * NW < nch)


def _wait(src, dst, sem):
    pltpu.make_async_copy(src, dst, sem).wait()


def _make_gather(dims, nch=NCHUNK):
    k = len(dims)
    trips = -(-nch // NW)
    scratch = []
    for d, dt in dims:
        scratch.append(pltpu.VMEM((trips, CH), jnp.int32))
        for b in range(2):
            scratch.append(pltpu.VMEM((CH, d), dt))
            scratch.append(pltpu.SemaphoreType.DMA)
            scratch.append(pltpu.SemaphoreType.DMA)
    out_type = [jax.ShapeDtypeStruct((nch * CH, d), dt) for (d, dt) in dims]

    @functools.partial(pl.kernel, out_type=out_type, mesh=_SC_MESH,
                       scratch_types=scratch)
    def gk(*refs):
        tabs = refs[0:2 * k:2]
        idx3 = refs[1:2 * k:2]
        outs = refs[2 * k:3 * k]
        scr = refs[3 * k:]
        idx_all = [scr[7 * j] for j in range(k)]
        rows = [[scr[7 * j + 1 + 3 * b] for b in range(2)] for j in range(k)]
        semg = [[scr[7 * j + 2 + 3 * b] for b in range(2)] for j in range(k)]
        semw = [[scr[7 * j + 3 + 3 * b] for b in range(2)] for j in range(k)]
        wid = lax.axis_index("s") * NC + lax.axis_index("c")
        for j in range(k):
            pltpu.sync_copy(idx3[j].at[:, wid], idx_all[j])

        def trip(t, b):
            tm1, tm2 = t - 1, t - 2
            for j in range(k):
                @pl.when(_validn(wid, tm2, nch))
                def _(j=j):
                    _wait(rows[j][b], outs[j].at[pl.ds(0, CH)], semw[j][b])
            for j in range(k):
                @pl.when(_validn(wid, t, nch))
                def _(j=j):
                    pltpu.async_copy(tabs[j].at[idx_all[j].at[t]],
                                     rows[j][b], semg[j][b])
            for j in range(k):
                @pl.when(_validn(wid, tm1, nch))
                def _(j=j):
                    bp = 1 - b
                    _wait(tabs[j].at[pl.ds(0, CH)], rows[j][bp], semg[j][bp])
                    base = (wid + tm1 * NW) * CH
                    pltpu.async_copy(rows[j][bp], outs[j].at[pl.ds(base, CH)],
                                     semw[j][bp])

        def pair(i, carry):
            trip(2 * i, 0)
            trip(2 * i + 1, 1)
            return carry

        lax.fori_loop(0, (trips + 3) // 2, pair, 0)

    return gk


def _make_geom_gather(nch=NCHUNK):
    trips = -(-nch // NW)
    scratch = [pltpu.VMEM((N_NODES * 4,), jnp.float32),
               pltpu.VMEM((trips, CH), jnp.int32),
               pltpu.VMEM((trips, CH), jnp.int32),
               pltpu.VMEM((CH, 128), jnp.float32),
               pltpu.VMEM((CH, 128), jnp.float32),
               pltpu.VMEM((CH,), jnp.float32),
               pltpu.SemaphoreType.DMA,
               pltpu.SemaphoreType.DMA,
               pltpu.SemaphoreType.DMA,
               pltpu.SemaphoreType.DMA]
    out_type = [jax.ShapeDtypeStruct((nch * CH, 128), jnp.float32),
                jax.ShapeDtypeStruct((nch * CH,), jnp.float32)]

    @functools.partial(pl.kernel, out_type=out_type, mesh=_SC_MESH,
                       scratch_types=scratch,
                       compiler_params=pltpu.CompilerParams(
                           needs_layout_passes=False))
    def gk(hn_tab, pos4, row3, col3, ghn_out, d2_out,
           pos_v, ridx_all, cidx_all, rows0, rows1, d2_v,
           semg0, semg1, semw0, semw1):
        rows = [rows0, rows1]
        semg = [semg0, semg1]
        semw = [semw0, semw1]
        wid = lax.axis_index("s") * NC + lax.axis_index("c")
        pltpu.sync_copy(pos4, pos_v)
        pltpu.sync_copy(row3.at[:, wid], ridx_all)
        pltpu.sync_copy(col3.at[:, wid], cidx_all)

        def trip(t, b):
            tm1, tm2 = t - 1, t - 2

            @pl.when(_validn(wid, tm2, nch))
            def _():
                _wait(rows[b], ghn_out.at[pl.ds(0, CH)], semw[b])

            @pl.when(_validn(wid, t, nch))
            def _():
                pltpu.async_copy(hn_tab.at[cidx_all.at[t]], rows[b], semg[b])
                for l in range(CH // 16):
                    ri = ridx_all[t, pl.ds(l * 16, 16)] * 4
                    ci = cidx_all[t, pl.ds(l * 16, 16)] * 4
                    dx = (plsc.load_gather(pos_v, [ri])
                          - plsc.load_gather(pos_v, [ci]))
                    dy = (plsc.load_gather(pos_v, [ri + 1])
                          - plsc.load_gather(pos_v, [ci + 1]))
                    dz = (plsc.load_gather(pos_v, [ri + 2])
                          - plsc.load_gather(pos_v, [ci + 2]))
                    d2_v[pl.ds(l * 16, 16)] = (dx * dx + dy * dy + dz * dz
                                               + 1e-8)
                pltpu.sync_copy(d2_v,
                                d2_out.at[pl.ds((wid + t * NW) * CH, CH)])

            @pl.when(_validn(wid, tm1, nch))
            def _():
                bp = 1 - b
                _wait(hn_tab.at[pl.ds(0, CH)], rows[bp], semg[bp])
                base = (wid + tm1 * NW) * CH
                pltpu.async_copy(rows[bp], ghn_out.at[pl.ds(base, CH)],
                                 semw[bp])

        def pair(i, carry):
            trip(2 * i, 0)
            trip(2 * i + 1, 1)
            return carry

        lax.fori_loop(0, (trips + 3) // 2, pair, 0)

    return gk


def _scatter_loop(halves, idx_all, accum, rows, semv, wid, ways, trips):
    def fire(t, b):
        chunk = wid + t * ways
        for v, lo, n in halves:
            @pl.when(jnp.logical_and(
                _valid(wid, t, ways),
                jnp.logical_and(chunk >= lo, chunk < lo + n)))
            def _(v=v, lo=lo):
                pltpu.async_copy(v.at[pl.ds((chunk - lo) * CH, CH)],
                                 rows[b], semv[b])

    fire(0, 0)
    fire(1, 1)

    def trip(t, b):
        @pl.when(_valid(wid, t, ways))
        def _():
            _wait(halves[0][0].at[pl.ds(0, CH)], rows[b], semv[b])
            pltpu.sync_copy(rows[b], accum.at[idx_all.at[t]], add=True)
        fire(t + 2, b)

    def pair(i, carry):
        trip(2 * i, 0)
        trip(2 * i + 1, 1)
        return carry

    lax.fori_loop(0, (trips + 1) // 2, pair, 0)


def _make_scatter1(nsrc):
    scratch = [pltpu.VMEM((TRIPS32, CH), jnp.int32),
               pltpu.VMEM((CH, 128), jnp.float32),
               pltpu.VMEM((CH, 128), jnp.float32),
               pltpu.SemaphoreType.DMA,
               pltpu.SemaphoreType.DMA,
               pltpu.VMEM_SHARED((N_PAD, 128), jnp.float32)]
    out_type = jax.ShapeDtypeStruct((NC, N_PAD, 128), jnp.float32)

    @functools.partial(pl.kernel, out_type=out_type, mesh=_SC_MESH,
                       scratch_types=scratch)
    def sk(*refs):
        vals = refs[:nsrc]
        idx3, zeros, out = refs[nsrc:nsrc + 3]
        idx_all, rows0, rows1, semv0, semv1, accum = refs[nsrc + 3:]
        if nsrc == 1:
            halves = [(vals[0], 0, NCHUNK)]
        else:
            halves = [(vals[0], 0, NCHUNK // 2),
                      (vals[1], NCHUNK // 2, NCHUNK // 2)]
        c = lax.axis_index("c")
        s = lax.axis_index("s")
        wid = s * NC + c
        base_n = s * NODE_SLICE
        pltpu.sync_copy(zeros.at[pl.ds(base_n, NODE_SLICE)],
                        accum.at[pl.ds(base_n, NODE_SLICE)])
        pltpu.sync_copy(idx3.at[:, wid], idx_all)
        plsc.subcore_barrier()
        _scatter_loop(halves, idx_all, accum, [rows0, rows1],
                      [semv0, semv1], wid, NW, TRIPS32)
        plsc.subcore_barrier()
        pltpu.sync_copy(accum.at[pl.ds(base_n, NODE_SLICE)],
                        out.at[c, pl.ds(base_n, NODE_SLICE)])

    return sk


def _make_scatter2():
    scratch = [pltpu.VMEM((TRIPS16, CH), jnp.int32),
               pltpu.VMEM((CH, 128), jnp.float32),
               pltpu.VMEM((CH, 128), jnp.float32),
               pltpu.SemaphoreType.DMA,
               pltpu.SemaphoreType.DMA,
               pltpu.VMEM_SHARED((N_PAD, 128), jnp.float32)]
    out_type = jax.ShapeDtypeStruct((NC, N_PAD, 128), jnp.float32)

    @functools.partial(pl.kernel, out_type=out_type, mesh=_SC_MESH,
                       scratch_types=scratch)
    def sk(vla, vlb, il3, vra, vrb, ir3, zeros, out, idx_all, rows0, rows1,
           semv0, semv1, accum):
        c = lax.axis_index("c")
        s = lax.axis_index("s")
        base_n = s * NODE_SLICE
        pltpu.sync_copy(zeros.at[pl.ds(base_n, NODE_SLICE)],
                        accum.at[pl.ds(base_n, NODE_SLICE)])

        def run(va, vb, idx3):
            pltpu.sync_copy(idx3.at[:, s], idx_all)
            plsc.subcore_barrier()
            _scatter_loop([(va, 0, NCHUNK // 2),
                           (vb, NCHUNK // 2, NCHUNK // 2)],
                          idx_all, accum, [rows0, rows1],
                          [semv0, semv1], s, NS, TRIPS16)
            plsc.subcore_barrier()
            pltpu.sync_copy(accum.at[pl.ds(base_n, NODE_SLICE)],
                            out.at[c, pl.ds(base_n, NODE_SLICE)])

        @pl.when(c == 0)
        def _():
            run(vla, vlb, il3)

        @pl.when(c == 1)
        def _():
            run(vra, vrb, ir3)

    return sk


def _full_spec(shape):
    return pl.BlockSpec(shape, lambda i: tuple(0 for _ in shape))


def _row_spec(block_rows, ncols):
    return pl.BlockSpec((block_rows, ncols), lambda i: (i, 0))


def _ln(x, g, b):
    m = jnp.mean(x, -1, keepdims=True)
    xc = x - m
    v = jnp.mean(xc * xc, -1, keepdims=True)
    return xc * jax.lax.rsqrt(v + 1e-5) * g + b


def _mm(x, w, b=None):
    y = jax.lax.dot_general(x.astype(jnp.bfloat16), w.astype(jnp.bfloat16),
                            (((1,), (0,)), ((), ())),
                            preferred_element_type=jnp.float32)
    if b is not None:
        y = y + b
    return y


def _node_pre_body(x_ref, wl, bl, w1, b1, w2, b2, wc, bc,
                   h_ref, hn_ref, cent_ref):
    x = x_ref[...]
    h = _mm(x, wl[...], bl[...])
    h_ref[...] = h
    t = jnp.maximum(_mm(h, w1[...], b1[...]), 0.0)
    hn_ref[...] = _mm(t, w2[...], b2[...])
    cent_ref[...] = _mm(h, wc[...], bc[...])


def _edge_nbe_body(d2_ref, ghn_ref, off, we, be, w1, b1, w2, b2,
                   wm, bm, msg_ref):
    ghn = ghn_ref[...]
    d = jnp.sqrt(d2_ref[...])
    step = CUTOFF / (NUM_GAUSS - 1)
    coeff = -0.5 / step ** 2
    diff = d - off[...]
    smear = jnp.exp(coeff * diff * diff)
    ea = _mm(smear, we[...], be[...])
    t = jnp.maximum(_mm(ea, w1[...], b1[...]), 0.0)
    he = _mm(t, w2[...], b2[...])
    msg_ref[...] = _mm(he * ghn, wm[...], bm[...])


def _node_post_body(h_ref, cent_ref, agg_ref, lng, lnb, wo, bo,
                    w1, b1, w2, b2, wc2, bc2,
                    h2_ref, hn2_ref, cent2_ref):
    out = cent_ref[...] + agg_ref[0] + agg_ref[1]
    out = _ln(out, lng[...], lnb[...])
    h2 = h_ref[...] + _mm(jnp.maximum(out, 0.0), wo[...], bo[...])
    h2_ref[...] = h2
    t = jnp.maximum(_mm(h2, w1[...], b1[...]), 0.0)
    hn2_ref[...] = _mm(t, w2[...], b2[...])
    cent2_ref[...] = _mm(h2, wc2[...], bc2[...])


def _edge_bond1_body(hb_ref, gl_ref, gr_ref,
                     wbl, wnl, w1l, b1l, w2l, b2l,
                     wbr, wnr, w1r, b1r, w2r, b2r,
                     wfl, bfl, wfr, bfr, ws, bs,
                     ml_ref, mr_ref, part_ref):
    hb = hb_ref[...]
    gl = gl_ref[...]
    gr = gr_ref[...]
    il = _mm(hb, wbl[...]) * _mm(gl, wnl[...])
    t = jnp.maximum(_mm(il, w1l[...], b1l[...]), 0.0)
    ml_ref[...] = _mm(t, w2l[...], b2l[...])
    ir = _mm(hb, wbr[...]) * _mm(gr, wnr[...])
    t = jnp.maximum(_mm(ir, w1r[...], b1r[...]), 0.0)
    mr_ref[...] = _mm(t, w2r[...], b2r[...])
    part_ref[...] = (_mm(gl, wfl[...], bfl[...]) + _mm(gr, wfr[...], bfr[...])
                     + _mm(hb, ws[...], bs[...])).astype(jnp.bfloat16)


def _edge_bond2_body(hb_ref, ga1_ref, gac_ref, parta_ref, partb_ref,
                     lng, lnb, wo, bo, w1, b1, w2, b2, wm, bm,
                     hb2_ref, msg2_ref):
    gac = gac_ref[...]
    ga2 = jax.lax.bitcast_convert_type(gac << 16, jnp.float32)
    ghn2 = jax.lax.bitcast_convert_type(
        gac & jnp.int32(-65536), jnp.float32)
    nblk_h = EH // TE_H
    part = jnp.where(pl.program_id(0) < nblk_h,
                     parta_ref[...], partb_ref[...])
    pre = ga1_ref[...] + ga2 + part.astype(jnp.float32)
    pre = _ln(pre, lng[...], lnb[...])
    hb2 = (hb_ref[...].astype(jnp.float32)
           + _mm(jnp.maximum(pre, 0.0), wo[...], bo[...]))
    hb2_ref[...] = hb2
    t = jnp.maximum(_mm(hb2, w1[...], b1[...]), 0.0)
    he2 = _mm(t, w2[...], b2[...])
    msg2_ref[...] = _mm(he2 * ghn2, wm[...], bm[...])


def _node_final_body(h2_ref, cent2_ref, agg_ref, lng, lnb, wo, bo,
                     h3_ref):
    out = cent2_ref[...] + agg_ref[0] + agg_ref[1]
    out = _ln(out, lng[...], lnb[...])
    h3_ref[...] = h2_ref[...] + _mm(jnp.maximum(out, 0.0), wo[...], bo[...])


def _tc_call(body, grid, in_arrs, in_specs, out_shapes, out_specs):
    return pl.pallas_call(
        body,
        grid=(grid,),
        in_specs=in_specs,
        out_specs=out_specs,
        out_shape=out_shapes,
    )(*in_arrs)


def _agg_spec():
    return pl.BlockSpec((2, TN, 128), lambda i: (0, i, 0))


NCH_H = NCHUNK // 2
EH = NCH_H * CH
TE_H = 4000

_gather_g12h = _make_geom_gather(NCH_H)
_gather_g3h = _make_gather(((128, jnp.float32), (128, jnp.float32)), NCH_H)
_gather_g4 = _make_gather(((128, jnp.float32), (128, jnp.int32)))
_scatter_s1 = _make_scatter1(2)
_scatter_s3 = _make_scatter1(1)
_scatter_s2 = _make_scatter2()


def _b2(v):
    return v.reshape(1, -1)


def _idx3(idx, ways, trips):
    pad = trips * ways * CH - idx.shape[0]
    return jnp.pad(idx, (0, pad)).reshape(trips, ways, CH)


def _idx3h(idx, lo):
    trips = -(-NCH_H // NW)
    return _idx3(idx[lo * CH:(lo + NCH_H) * CH], NW, trips)


def kernel(h_node, pos_node, h_bond, bond_index, batch, is_mol, is_frag, params):
    P = params
    nbe = P["nbe"][0]
    nbb = P["nbb"][0]
    bb = P["bb"][0]
    row = bond_index[0]
    col = bond_index[1]
    row32 = _idx3(row, NW, TRIPS32)
    col32 = _idx3(col, NW, TRIPS32)
    row32a, row32b = _idx3h(row, 0), _idx3h(row, NCH_H)
    col32a, col32b = _idx3h(col, 0), _idx3h(col, NCH_H)
    row16 = _idx3(row, NS, TRIPS16)
    col16 = _idx3(col, NS, TRIPS16)
    zeros_n = jnp.zeros((N_PAD, 128), jnp.float32)

    pos4 = jnp.zeros((N_NODES, 4), jnp.float32).at[:, :3].set(pos_node)
    pos4 = pos4.reshape(N_NODES * 4)
    off = np.zeros((1, 32), np.float32)
    off[0, :NUM_GAUSS] = np.linspace(0.0, CUTOFF, NUM_GAUSS)
    off = jnp.asarray(off)
    we_pad = jnp.zeros((32, 128), jnp.float32).at[:NUM_GAUSS].set(P["edge_emb"]["W"])

    ew = _full_spec

    h, hn_all, cent = _tc_call(
        _node_pre_body, N_NODES // TN,
        [h_node, P["lin_node"]["W"], _b2(P["lin_node"]["b"]),
         nbe["node_net"]["l1"]["W"], _b2(nbe["node_net"]["l1"]["b"]),
         nbe["node_net"]["l2"]["W"], _b2(nbe["node_net"]["l2"]["b"]),
         nbe["centroid"]["W"], _b2(nbe["centroid"]["b"])],
        [_row_spec(TN, 128)] + [ew((128, 128)), ew((1, 128))] * 4,
        [jax.ShapeDtypeStruct((N_NODES, 128), jnp.float32)] * 3,
        [_row_spec(TN, 128)] * 3,
    )

    g12 = [_gather_g12h(hn_all, pos4, r3, c3)
           for r3, c3 in ((row32a, col32a), (row32b, col32b))]
    msg_h = []
    for ghn, d2 in g12:
        d2 = d2.reshape(EH, 1)
        (m,) = _tc_call(
            _edge_nbe_body, EH // TE_H,
            [d2, ghn, off, we_pad, _b2(P["edge_emb"]["b"]),
             nbe["edge_net"]["l1"]["W"], _b2(nbe["edge_net"]["l1"]["b"]),
             nbe["edge_net"]["l2"]["W"], _b2(nbe["edge_net"]["l2"]["b"]),
             nbe["msg_net"]["W"], _b2(nbe["msg_net"]["b"])],
            [_row_spec(TE_H, 1), _row_spec(TE_H, 128),
             ew((1, 32)), ew((32, 128)), ew((1, 128)),
             ew((128, 128)), ew((1, 128)), ew((128, 128)), ew((1, 128)),
             ew((128, 128)), ew((1, 128))],
            [jax.ShapeDtypeStruct((EH, 128), jnp.float32)],
            [_row_spec(TE_H, 128)],
        )
        msg_h.append(m)

    aggr = _scatter_s1(msg_h[0], msg_h[1], row32, zeros_n)

    h2, hn2, cent2 = _tc_call(
        _node_post_body, N_NODES // TN,
        [h, cent, aggr, _b2(nbe["ln_g"]), _b2(nbe["ln_b"]),
         nbe["out"]["W"], _b2(nbe["out"]["b"]),
         nbb["node_net"]["l1"]["W"], _b2(nbb["node_net"]["l1"]["b"]),
         nbb["node_net"]["l2"]["W"], _b2(nbb["node_net"]["l2"]["b"]),
         nbb["centroid"]["W"], _b2(nbb["centroid"]["b"])],
        [_row_spec(TN, 128)] * 2 + [_agg_spec()]
        + [ew((1, 128)), ew((1, 128))]
        + [ew((128, 128)), ew((1, 128))] * 4,
        [jax.ShapeDtypeStruct((N_NODES, 128), jnp.float32)] * 3,
        [_row_spec(TN, 128)] * 3,
    )

    hb16 = h_bond.astype(jnp.bfloat16)
    ml_h, mr_h, part_h = [], [], []
    for hidx, (r3, c3) in enumerate(((row32a, col32a), (row32b, col32b))):
        gl, gr = _gather_g3h(h2, r3, h2, c3)
        hb_spec = pl.BlockSpec((TE_H, 128),
                               lambda i, H=hidx: (i + H * (EH // TE_H), 0))
        ml, mr, pt = _tc_call(
            _edge_bond1_body, EH // TE_H,
            [hb16, gl, gr,
             bb["ffn_l"]["bond_lin"]["W"], bb["ffn_l"]["node_lin"]["W"],
             bb["ffn_l"]["inter"]["l1"]["W"], _b2(bb["ffn_l"]["inter"]["l1"]["b"]),
             bb["ffn_l"]["inter"]["l2"]["W"], _b2(bb["ffn_l"]["inter"]["l2"]["b"]),
             bb["ffn_r"]["bond_lin"]["W"], bb["ffn_r"]["node_lin"]["W"],
             bb["ffn_r"]["inter"]["l1"]["W"], _b2(bb["ffn_r"]["inter"]["l1"]["b"]),
             bb["ffn_r"]["inter"]["l2"]["W"], _b2(bb["ffn_r"]["inter"]["l2"]["b"]),
             bb["node_ffn_l"]["W"], _b2(bb["node_ffn_l"]["b"]),
             bb["node_ffn_r"]["W"], _b2(bb["node_ffn_r"]["b"]),
             bb["self_ffn"]["W"], _b2(bb["self_ffn"]["b"])],
            [hb_spec, _row_spec(TE_H, 128), _row_spec(TE_H, 128)]
            + [ew((128, 256)), ew((128, 256)), ew((256, 256)), ew((1, 256)),
               ew((256, 128)), ew((1, 128))] * 2
            + [ew((128, 128)), ew((1, 128))] * 3,
            [jax.ShapeDtypeStruct((EH, 128), jnp.float32)] * 2
            + [jax.ShapeDtypeStruct((EH, 128), jnp.bfloat16)],
            [_row_spec(TE_H, 128)] * 3,
        )
        ml_h.append(ml)
        mr_h.append(mr)
        part_h.append(pt)

    A12 = _scatter_s2(ml_h[0], ml_h[1], col16, mr_h[0], mr_h[1], row16,
                      zeros_n)
    A1 = A12[0]
    a2b = jax.lax.bitcast_convert_type(A12[1][:N_NODES], jnp.uint32)
    hnb = jax.lax.bitcast_convert_type(hn2, jnp.uint32)
    tac = jax.lax.bitcast_convert_type(
        (hnb & jnp.uint32(0xFFFF0000)) | (a2b >> 16), jnp.int32)

    gA1, gac = _gather_g4(A1, row32, tac, col32)

    nblk_h = EH // TE_H
    parta_spec = pl.BlockSpec((TE_H, 128),
                              lambda i: (jnp.minimum(i, nblk_h - 1), 0))
    partb_spec = pl.BlockSpec((TE_H, 128),
                              lambda i: (jnp.maximum(i - nblk_h, 0), 0))
    hb2, msg2 = _tc_call(
        _edge_bond2_body, N_EDGES // TE_H,
        [hb16, gA1, gac, part_h[0], part_h[1],
         _b2(bb["ln_g"]), _b2(bb["ln_b"]),
         bb["out"]["W"], _b2(bb["out"]["b"]),
         nbb["edge_net"]["l1"]["W"], _b2(nbb["edge_net"]["l1"]["b"]),
         nbb["edge_net"]["l2"]["W"], _b2(nbb["edge_net"]["l2"]["b"]),
         nbb["msg_net"]["W"], _b2(nbb["msg_net"]["b"])],
        [_row_spec(TE_H, 128)] * 3 + [parta_spec, partb_spec]
        + [ew((1, 128)), ew((1, 128))]
        + [ew((128, 128)), ew((1, 128))] * 4,
        [jax.ShapeDtypeStruct((N_EDGES, 128), jnp.float32)] * 2,
        [_row_spec(TE_H, 128)] * 2,
    )

    aggr2 = _scatter_s3(msg2, row32, zeros_n)

    (h3,) = _tc_call(
        _node_final_body, N_NODES // TN,
        [h2, cent2, aggr2, _b2(nbb["ln_g"]), _b2(nbb["ln_b"]),
         nbb["out"]["W"], _b2(nbb["out"]["b"])],
        [_row_spec(TN, 128)] * 2 + [_agg_spec()]
        + [ew((1, 128)), ew((1, 128)), ew((128, 128)), ew((1, 128))],
        [jax.ShapeDtypeStruct((N_NODES, 128), jnp.float32)],
        [_row_spec(TN, 128)],
    )

    return h3, hb2
